# Initial kernel scaffold; baseline (speedup 1.0000x reference)
import jax
import jax.numpy as jnp
from jax.experimental import pallas as pl


def kernel(x, adj_matrix, params):
    raise NotImplementedError("write your pallas kernel here")


# speedup vs baseline: 8.3766x; 8.3766x over previous
import functools

import jax
import jax.numpy as jnp
from jax.experimental import pallas as pl
from jax.experimental.pallas import tpu as pltpu

INPUT = 256
HIDDEN = 128
B = 8
N = 1024
BC = 256
C = 128
K = N // C
G4 = 4 * HIDDEN


def _agg_kernel(adj_ref, xf_ref, xr_ref, out_ref):
    m = (adj_ref[0] > 0).astype(jnp.float32)
    nsum = jnp.dot(m, xf_ref[0], preferred_element_type=jnp.float32)
    deg = jnp.sum(m, axis=1, keepdims=True)
    out_ref[0] = (xr_ref[0] + nsum) / (1.0 + deg)


def _proj0_kernel(feat_ref, w_ref, b_ref, out_ref):
    w = w_ref[...]
    b = b_ref[...]
    for bb in range(B):
        out_ref[:, bb, :] = (
            jnp.dot(feat_ref[bb], w, preferred_element_type=jnp.float32) + b
        )


def _proj1_kernel(hf_ref, hb_ref, wa_ref, wb_ref, b_ref, out_ref):
    wa = wa_ref[...]
    wb = wb_ref[...]
    b = b_ref[...]
    for bb in range(B):
        gf = jnp.dot(hf_ref[:, bb, :], wa, preferred_element_type=jnp.float32)
        gb = jnp.dot(hb_ref[:, bb, :], wb, preferred_element_type=jnp.float32)
        out_ref[:, bb, :] = gf + gb + b


def _lstm_cell(gates, c_old):
    i = jax.nn.sigmoid(gates[:, 0 * HIDDEN:1 * HIDDEN])
    f = jax.nn.sigmoid(gates[:, 1 * HIDDEN:2 * HIDDEN])
    g = jnp.tanh(gates[:, 2 * HIDDEN:3 * HIDDEN])
    o = jax.nn.sigmoid(gates[:, 3 * HIDDEN:4 * HIDDEN])
    c_new = f * c_old + i * g
    h_new = o * jnp.tanh(c_new)
    return h_new, c_new


def _scan0_kernel(gf_ref, gb_ref, w_ref, hsf_ref, hsb_ref, h_ref, c_ref):
    i = pl.program_id(0)

    @pl.when(i == 0)
    def _init():
        h_ref[...] = jnp.zeros_like(h_ref)
        c_ref[...] = jnp.zeros_like(c_ref)

    def step(j, _):
        h = h_ref[...]
        gates = jnp.dot(h, w_ref[...], preferred_element_type=jnp.float32)
        gates_f = gates[:, :G4] + gf_ref[j]
        gates_b = gates[:, G4:] + gb_ref[C - 1 - j]
        c = c_ref[...]
        hf, cf = _lstm_cell(gates_f, c[:, :HIDDEN])
        hb, cb = _lstm_cell(gates_b, c[:, HIDDEN:])
        h_ref[:, :HIDDEN] = hf
        h_ref[:, HIDDEN:] = hb
        c_ref[:, :HIDDEN] = cf
        c_ref[:, HIDDEN:] = cb
        hsf_ref[j] = hf
        hsb_ref[C - 1 - j] = hb
        return 0

    jax.lax.fori_loop(0, C, step, 0)


def _scan1_kernel(gf_ref, gb_ref, w_ref, out_ref, h_ref, c_ref):
    i = pl.program_id(0)

    @pl.when(i == 0)
    def _init():
        h_ref[...] = jnp.zeros_like(h_ref)
        c_ref[...] = jnp.zeros_like(c_ref)

    def step(j, _):
        h = h_ref[...]
        gates = jnp.dot(h, w_ref[...], preferred_element_type=jnp.float32)
        gates_f = gates[:, :G4] + gf_ref[j]
        gates_b = gates[:, G4:] + gb_ref[C - 1 - j]
        c = c_ref[...]
        hf, cf = _lstm_cell(gates_f, c[:, :HIDDEN])
        hb, cb = _lstm_cell(gates_b, c[:, HIDDEN:])
        h_ref[:, :HIDDEN] = hf
        h_ref[:, HIDDEN:] = hb
        c_ref[:, :HIDDEN] = cf
        c_ref[:, HIDDEN:] = cb
        return 0

    jax.lax.fori_loop(0, C, step, 0)
    out_ref[...] = h_ref[...]


def _block_diag_hh(whh_f, whh_b):
    z = jnp.zeros((HIDDEN, G4), jnp.float32)
    top = jnp.concatenate([whh_f.T, z], axis=1)
    bot = jnp.concatenate([z, whh_b.T], axis=1)
    return jnp.concatenate([top, bot], axis=0)


@jax.jit
def _run(x, adj_matrix, params):
    f32 = jnp.float32
    x = x.astype(f32)

    w0 = jnp.concatenate(
        [params["W_ih_l0_d0"].T, params["W_ih_l0_d1"].T], axis=1)
    b0 = (params["b_ih_l0_d0"] + params["b_hh_l0_d0"],
          params["b_ih_l0_d1"] + params["b_hh_l0_d1"])
    b0 = jnp.concatenate(b0, axis=0).reshape(1, 2 * G4)
    whh0 = _block_diag_hh(params["W_hh_l0_d0"], params["W_hh_l0_d1"])

    w1 = jnp.concatenate(
        [params["W_ih_l1_d0"].T, params["W_ih_l1_d1"].T], axis=1)
    w1a = w1[:HIDDEN]
    w1b = w1[HIDDEN:]
    b1 = (params["b_ih_l1_d0"] + params["b_hh_l1_d0"],
          params["b_ih_l1_d1"] + params["b_hh_l1_d1"])
    b1 = jnp.concatenate(b1, axis=0).reshape(1, 2 * G4)
    whh1 = _block_diag_hh(params["W_hh_l1_d0"], params["W_hh_l1_d1"])

    feat = pl.pallas_call(
        _agg_kernel,
        grid=(B, N // BC),
        in_specs=[
            pl.BlockSpec((1, BC, N), lambda b, i: (b, i, 0)),
            pl.BlockSpec((1, N, INPUT), lambda b, i: (b, 0, 0)),
            pl.BlockSpec((1, BC, INPUT), lambda b, i: (b, i, 0)),
        ],
        out_specs=pl.BlockSpec((1, BC, INPUT), lambda b, i: (b, i, 0)),
        out_shape=jax.ShapeDtypeStruct((B, N, INPUT), f32),
    )(adj_matrix, x, x)

    g0 = pl.pallas_call(
        _proj0_kernel,
        grid=(K,),
        in_specs=[
            pl.BlockSpec((B, C, INPUT), lambda i: (0, i, 0)),
            pl.BlockSpec((INPUT, 2 * G4), lambda i: (0, 0)),
            pl.BlockSpec((1, 2 * G4), lambda i: (0, 0)),
        ],
        out_specs=pl.BlockSpec((C, B, 2 * G4), lambda i: (i, 0, 0)),
        out_shape=jax.ShapeDtypeStruct((N, B, 2 * G4), f32),
    )(feat, w0, b0)

    hs_f, hs_b = pl.pallas_call(
        _scan0_kernel,
        grid=(K,),
        in_specs=[
            pl.BlockSpec((C, B, G4), lambda i: (i, 0, 0)),
            pl.BlockSpec((C, B, G4), lambda i: (K - 1 - i, 0, 1)),
            pl.BlockSpec((2 * HIDDEN, 2 * G4), lambda i: (0, 0)),
        ],
        out_specs=[
            pl.BlockSpec((C, B, HIDDEN), lambda i: (i, 0, 0)),
            pl.BlockSpec((C, B, HIDDEN), lambda i: (K - 1 - i, 0, 0)),
        ],
        out_shape=[
            jax.ShapeDtypeStruct((N, B, HIDDEN), f32),
            jax.ShapeDtypeStruct((N, B, HIDDEN), f32),
        ],
        scratch_shapes=[
            pltpu.VMEM((B, 2 * HIDDEN), f32),
            pltpu.VMEM((B, 2 * HIDDEN), f32),
        ],
    )(g0, g0, whh0)

    g1 = pl.pallas_call(
        _proj1_kernel,
        grid=(K,),
        in_specs=[
            pl.BlockSpec((C, B, HIDDEN), lambda i: (i, 0, 0)),
            pl.BlockSpec((C, B, HIDDEN), lambda i: (i, 0, 0)),
            pl.BlockSpec((HIDDEN, 2 * G4), lambda i: (0, 0)),
            pl.BlockSpec((HIDDEN, 2 * G4), lambda i: (0, 0)),
            pl.BlockSpec((1, 2 * G4), lambda i: (0, 0)),
        ],
        out_specs=pl.BlockSpec((C, B, 2 * G4), lambda i: (i, 0, 0)),
        out_shape=jax.ShapeDtypeStruct((N, B, 2 * G4), f32),
    )(hs_f, hs_b, w1a, w1b, b1)

    out = pl.pallas_call(
        _scan1_kernel,
        grid=(K,),
        in_specs=[
            pl.BlockSpec((C, B, G4), lambda i: (i, 0, 0)),
            pl.BlockSpec((C, B, G4), lambda i: (K - 1 - i, 0, 1)),
            pl.BlockSpec((2 * HIDDEN, 2 * G4), lambda i: (0, 0)),
        ],
        out_specs=pl.BlockSpec((B, 2 * HIDDEN), lambda i: (0, 0)),
        out_shape=jax.ShapeDtypeStruct((B, 2 * HIDDEN), f32),
        scratch_shapes=[
            pltpu.VMEM((B, 2 * HIDDEN), f32),
            pltpu.VMEM((B, 2 * HIDDEN), f32),
        ],
    )(g1, g1, whh1)

    return out


def kernel(x, adj_matrix, params):
    return _run(x, adj_matrix, params)


# speedup vs baseline: 9.5969x; 1.1457x over previous
import functools

import jax
import jax.numpy as jnp
from jax.experimental import pallas as pl
from jax.experimental.pallas import tpu as pltpu

INPUT = 256
HIDDEN = 128
B = 8
N = 1024
BC = 256
C = 128
K = N // C
G4 = 4 * HIDDEN


def _agg_kernel(adj_ref, xf_ref, xr_ref, out_ref):
    m = (adj_ref[0] > 0).astype(jnp.float32)
    nsum = jnp.dot(m, xf_ref[0], preferred_element_type=jnp.float32)
    deg = jnp.sum(m, axis=1, keepdims=True)
    out_ref[0] = (xr_ref[0] + nsum) / (1.0 + deg)


def _proj0_kernel(feat_ref, w_ref, b_ref, out_ref):
    w = w_ref[...]
    b = b_ref[...]
    for bb in range(B):
        out_ref[:, bb, :] = (
            jnp.dot(feat_ref[bb], w, preferred_element_type=jnp.float32) + b
        )


def _proj1_kernel(hf_ref, hb_ref, wa_ref, wb_ref, b_ref, out_ref):
    wa = wa_ref[...]
    wb = wb_ref[...]
    b = b_ref[...]
    for bb in range(B):
        gf = jnp.dot(hf_ref[:, bb, :], wa, preferred_element_type=jnp.float32)
        gb = jnp.dot(hb_ref[:, bb, :], wb, preferred_element_type=jnp.float32)
        out_ref[:, bb, :] = gf + gb + b


def _lstm_cell(gates, c_old):
    i = jax.nn.sigmoid(gates[:, 0 * HIDDEN:1 * HIDDEN])
    f = jax.nn.sigmoid(gates[:, 1 * HIDDEN:2 * HIDDEN])
    g = jnp.tanh(gates[:, 2 * HIDDEN:3 * HIDDEN])
    o = jax.nn.sigmoid(gates[:, 3 * HIDDEN:4 * HIDDEN])
    c_new = f * c_old + i * g
    h_new = o * jnp.tanh(c_new)
    return h_new, c_new


def _scan0_kernel(gf_ref, gb_ref, wf_ref, wb_ref, hsf_ref, hsb_ref,
                  h_ref, c_ref):
    i = pl.program_id(0)

    @pl.when(i == 0)
    def _init():
        h_ref[...] = jnp.zeros_like(h_ref)
        c_ref[...] = jnp.zeros_like(c_ref)

    def step(j, _):
        h = h_ref[...]
        gates_f = jnp.dot(h[:, :HIDDEN], wf_ref[...],
                          preferred_element_type=jnp.float32) + gf_ref[j]
        gates_b = jnp.dot(h[:, HIDDEN:], wb_ref[...],
                          preferred_element_type=jnp.float32) + gb_ref[C - 1 - j]
        c = c_ref[...]
        hf, cf = _lstm_cell(gates_f, c[:, :HIDDEN])
        hb, cb = _lstm_cell(gates_b, c[:, HIDDEN:])
        h_ref[:, :HIDDEN] = hf
        h_ref[:, HIDDEN:] = hb
        c_ref[:, :HIDDEN] = cf
        c_ref[:, HIDDEN:] = cb
        hsf_ref[j] = hf
        hsb_ref[C - 1 - j] = hb
        return 0

    jax.lax.fori_loop(0, C, step, 0)


def _scan1_kernel(gf_ref, gb_ref, wf_ref, wb_ref, out_ref, h_ref, c_ref):
    i = pl.program_id(0)

    @pl.when(i == 0)
    def _init():
        h_ref[...] = jnp.zeros_like(h_ref)
        c_ref[...] = jnp.zeros_like(c_ref)

    def step(j, _):
        h = h_ref[...]
        gates_f = jnp.dot(h[:, :HIDDEN], wf_ref[...],
                          preferred_element_type=jnp.float32) + gf_ref[j]
        gates_b = jnp.dot(h[:, HIDDEN:], wb_ref[...],
                          preferred_element_type=jnp.float32) + gb_ref[C - 1 - j]
        c = c_ref[...]
        hf, cf = _lstm_cell(gates_f, c[:, :HIDDEN])
        hb, cb = _lstm_cell(gates_b, c[:, HIDDEN:])
        h_ref[:, :HIDDEN] = hf
        h_ref[:, HIDDEN:] = hb
        c_ref[:, :HIDDEN] = cf
        c_ref[:, HIDDEN:] = cb
        return 0

    jax.lax.fori_loop(0, C, step, 0)
    out_ref[...] = h_ref[...]


@jax.jit
def _run(x, adj_matrix, params):
    f32 = jnp.float32
    x = x.astype(f32)

    w0 = jnp.concatenate(
        [params["W_ih_l0_d0"].T, params["W_ih_l0_d1"].T], axis=1)
    b0 = (params["b_ih_l0_d0"] + params["b_hh_l0_d0"],
          params["b_ih_l0_d1"] + params["b_hh_l0_d1"])
    b0 = jnp.concatenate(b0, axis=0).reshape(1, 2 * G4)
    whh0f = params["W_hh_l0_d0"].T
    whh0b = params["W_hh_l0_d1"].T

    w1 = jnp.concatenate(
        [params["W_ih_l1_d0"].T, params["W_ih_l1_d1"].T], axis=1)
    w1a = w1[:HIDDEN]
    w1b = w1[HIDDEN:]
    b1 = (params["b_ih_l1_d0"] + params["b_hh_l1_d0"],
          params["b_ih_l1_d1"] + params["b_hh_l1_d1"])
    b1 = jnp.concatenate(b1, axis=0).reshape(1, 2 * G4)
    whh1f = params["W_hh_l1_d0"].T
    whh1b = params["W_hh_l1_d1"].T

    feat = pl.pallas_call(
        _agg_kernel,
        grid=(B, N // BC),
        in_specs=[
            pl.BlockSpec((1, BC, N), lambda b, i: (b, i, 0)),
            pl.BlockSpec((1, N, INPUT), lambda b, i: (b, 0, 0)),
            pl.BlockSpec((1, BC, INPUT), lambda b, i: (b, i, 0)),
        ],
        out_specs=pl.BlockSpec((1, BC, INPUT), lambda b, i: (b, i, 0)),
        out_shape=jax.ShapeDtypeStruct((B, N, INPUT), f32),
    )(adj_matrix, x, x)

    g0 = pl.pallas_call(
        _proj0_kernel,
        grid=(K,),
        in_specs=[
            pl.BlockSpec((B, C, INPUT), lambda i: (0, i, 0)),
            pl.BlockSpec((INPUT, 2 * G4), lambda i: (0, 0)),
            pl.BlockSpec((1, 2 * G4), lambda i: (0, 0)),
        ],
        out_specs=pl.BlockSpec((C, B, 2 * G4), lambda i: (i, 0, 0)),
        out_shape=jax.ShapeDtypeStruct((N, B, 2 * G4), f32),
    )(feat, w0, b0)

    hs_f, hs_b = pl.pallas_call(
        _scan0_kernel,
        grid=(K,),
        in_specs=[
            pl.BlockSpec((C, B, G4), lambda i: (i, 0, 0)),
            pl.BlockSpec((C, B, G4), lambda i: (K - 1 - i, 0, 1)),
            pl.BlockSpec((HIDDEN, G4), lambda i: (0, 0)),
            pl.BlockSpec((HIDDEN, G4), lambda i: (0, 0)),
        ],
        out_specs=[
            pl.BlockSpec((C, B, HIDDEN), lambda i: (i, 0, 0)),
            pl.BlockSpec((C, B, HIDDEN), lambda i: (K - 1 - i, 0, 0)),
        ],
        out_shape=[
            jax.ShapeDtypeStruct((N, B, HIDDEN), f32),
            jax.ShapeDtypeStruct((N, B, HIDDEN), f32),
        ],
        scratch_shapes=[
            pltpu.VMEM((B, 2 * HIDDEN), f32),
            pltpu.VMEM((B, 2 * HIDDEN), f32),
        ],
    )(g0, g0, whh0f, whh0b)

    g1 = pl.pallas_call(
        _proj1_kernel,
        grid=(K,),
        in_specs=[
            pl.BlockSpec((C, B, HIDDEN), lambda i: (i, 0, 0)),
            pl.BlockSpec((C, B, HIDDEN), lambda i: (i, 0, 0)),
            pl.BlockSpec((HIDDEN, 2 * G4), lambda i: (0, 0)),
            pl.BlockSpec((HIDDEN, 2 * G4), lambda i: (0, 0)),
            pl.BlockSpec((1, 2 * G4), lambda i: (0, 0)),
        ],
        out_specs=pl.BlockSpec((C, B, 2 * G4), lambda i: (i, 0, 0)),
        out_shape=jax.ShapeDtypeStruct((N, B, 2 * G4), f32),
    )(hs_f, hs_b, w1a, w1b, b1)

    out = pl.pallas_call(
        _scan1_kernel,
        grid=(K,),
        in_specs=[
            pl.BlockSpec((C, B, G4), lambda i: (i, 0, 0)),
            pl.BlockSpec((C, B, G4), lambda i: (K - 1 - i, 0, 1)),
            pl.BlockSpec((HIDDEN, G4), lambda i: (0, 0)),
            pl.BlockSpec((HIDDEN, G4), lambda i: (0, 0)),
        ],
        out_specs=pl.BlockSpec((B, 2 * HIDDEN), lambda i: (0, 0)),
        out_shape=jax.ShapeDtypeStruct((B, 2 * HIDDEN), f32),
        scratch_shapes=[
            pltpu.VMEM((B, 2 * HIDDEN), f32),
            pltpu.VMEM((B, 2 * HIDDEN), f32),
        ],
    )(g1, g1, whh1f, whh1b)

    return out


def kernel(x, adj_matrix, params):
    return _run(x, adj_matrix, params)


# speedup vs baseline: 9.6803x; 1.0087x over previous
import functools

import jax
import jax.numpy as jnp
from jax.experimental import pallas as pl
from jax.experimental.pallas import tpu as pltpu

INPUT = 256
HIDDEN = 128
B = 8
N = 1024
BC = 256
C = 128
K = N // C
G4 = 4 * HIDDEN


def _agg_kernel(adj_ref, xf_ref, xr_ref, out_ref):
    m = (adj_ref[0] > 0).astype(jnp.float32)
    nsum = jnp.dot(m, xf_ref[0], preferred_element_type=jnp.float32)
    deg = jnp.sum(m, axis=1, keepdims=True)
    out_ref[0] = (xr_ref[0] + nsum) / (1.0 + deg)


def _proj0_kernel(feat_ref, w_ref, b_ref, out_ref):
    w = w_ref[...]
    b = b_ref[...]
    for bb in range(B):
        out_ref[:, bb, :] = (
            jnp.dot(feat_ref[bb], w, preferred_element_type=jnp.float32) + b
        )


def _proj1_kernel(hf_ref, hb_ref, wa_ref, wb_ref, b_ref, out_ref):
    wa = wa_ref[...]
    wb = wb_ref[...]
    b = b_ref[...]
    for bb in range(B):
        gf = jnp.dot(hf_ref[:, bb, :], wa, preferred_element_type=jnp.float32)
        gb = jnp.dot(hb_ref[:, bb, :], wb, preferred_element_type=jnp.float32)
        out_ref[:, bb, :] = gf + gb + b


def _lstm_cell(gates, c_old):
    i = jax.nn.sigmoid(gates[:, 0 * HIDDEN:1 * HIDDEN])
    f = jax.nn.sigmoid(gates[:, 1 * HIDDEN:2 * HIDDEN])
    g = jnp.tanh(gates[:, 2 * HIDDEN:3 * HIDDEN])
    o = jax.nn.sigmoid(gates[:, 3 * HIDDEN:4 * HIDDEN])
    c_new = f * c_old + i * g
    h_new = o * jnp.tanh(c_new)
    return h_new, c_new


def _scan0_kernel(gf_ref, gb_ref, wf_ref, wb_ref, hsf_ref, hsb_ref,
                  h_ref, c_ref):
    i = pl.program_id(0)

    @pl.when(i == 0)
    def _init():
        h_ref[...] = jnp.zeros_like(h_ref)
        c_ref[...] = jnp.zeros_like(c_ref)

    def step(j, _):
        h = h_ref[...]
        hb16 = h.astype(jnp.bfloat16)
        gates_f = jnp.dot(hb16[:, :HIDDEN], wf_ref[...],
                          preferred_element_type=jnp.float32) + gf_ref[j]
        gates_b = jnp.dot(hb16[:, HIDDEN:], wb_ref[...],
                          preferred_element_type=jnp.float32) + gb_ref[C - 1 - j]
        c = c_ref[...]
        hf, cf = _lstm_cell(gates_f, c[:, :HIDDEN])
        hb, cb = _lstm_cell(gates_b, c[:, HIDDEN:])
        h_ref[:, :HIDDEN] = hf
        h_ref[:, HIDDEN:] = hb
        c_ref[:, :HIDDEN] = cf
        c_ref[:, HIDDEN:] = cb
        hsf_ref[j] = hf
        hsb_ref[C - 1 - j] = hb
        return 0

    jax.lax.fori_loop(0, C, step, 0)


def _scan1_kernel(gf_ref, gb_ref, wf_ref, wb_ref, out_ref, h_ref, c_ref):
    i = pl.program_id(0)

    @pl.when(i == 0)
    def _init():
        h_ref[...] = jnp.zeros_like(h_ref)
        c_ref[...] = jnp.zeros_like(c_ref)

    def step(j, _):
        h = h_ref[...]
        hb16 = h.astype(jnp.bfloat16)
        gates_f = jnp.dot(hb16[:, :HIDDEN], wf_ref[...],
                          preferred_element_type=jnp.float32) + gf_ref[j]
        gates_b = jnp.dot(hb16[:, HIDDEN:], wb_ref[...],
                          preferred_element_type=jnp.float32) + gb_ref[C - 1 - j]
        c = c_ref[...]
        hf, cf = _lstm_cell(gates_f, c[:, :HIDDEN])
        hb, cb = _lstm_cell(gates_b, c[:, HIDDEN:])
        h_ref[:, :HIDDEN] = hf
        h_ref[:, HIDDEN:] = hb
        c_ref[:, :HIDDEN] = cf
        c_ref[:, HIDDEN:] = cb
        return 0

    jax.lax.fori_loop(0, C, step, 0)
    out_ref[...] = h_ref[...]


@jax.jit
def _run(x, adj_matrix, params):
    f32 = jnp.float32
    x = x.astype(f32)

    w0 = jnp.concatenate(
        [params["W_ih_l0_d0"].T, params["W_ih_l0_d1"].T], axis=1)
    b0 = (params["b_ih_l0_d0"] + params["b_hh_l0_d0"],
          params["b_ih_l0_d1"] + params["b_hh_l0_d1"])
    b0 = jnp.concatenate(b0, axis=0).reshape(1, 2 * G4)
    whh0f = params["W_hh_l0_d0"].T.astype(jnp.bfloat16)
    whh0b = params["W_hh_l0_d1"].T.astype(jnp.bfloat16)

    w1 = jnp.concatenate(
        [params["W_ih_l1_d0"].T, params["W_ih_l1_d1"].T], axis=1)
    w1a = w1[:HIDDEN]
    w1b = w1[HIDDEN:]
    b1 = (params["b_ih_l1_d0"] + params["b_hh_l1_d0"],
          params["b_ih_l1_d1"] + params["b_hh_l1_d1"])
    b1 = jnp.concatenate(b1, axis=0).reshape(1, 2 * G4)
    whh1f = params["W_hh_l1_d0"].T.astype(jnp.bfloat16)
    whh1b = params["W_hh_l1_d1"].T.astype(jnp.bfloat16)

    feat = pl.pallas_call(
        _agg_kernel,
        grid=(B, N // BC),
        in_specs=[
            pl.BlockSpec((1, BC, N), lambda b, i: (b, i, 0)),
            pl.BlockSpec((1, N, INPUT), lambda b, i: (b, 0, 0)),
            pl.BlockSpec((1, BC, INPUT), lambda b, i: (b, i, 0)),
        ],
        out_specs=pl.BlockSpec((1, BC, INPUT), lambda b, i: (b, i, 0)),
        out_shape=jax.ShapeDtypeStruct((B, N, INPUT), f32),
    )(adj_matrix, x, x)

    g0 = pl.pallas_call(
        _proj0_kernel,
        grid=(K,),
        in_specs=[
            pl.BlockSpec((B, C, INPUT), lambda i: (0, i, 0)),
            pl.BlockSpec((INPUT, 2 * G4), lambda i: (0, 0)),
            pl.BlockSpec((1, 2 * G4), lambda i: (0, 0)),
        ],
        out_specs=pl.BlockSpec((C, B, 2 * G4), lambda i: (i, 0, 0)),
        out_shape=jax.ShapeDtypeStruct((N, B, 2 * G4), f32),
    )(feat, w0, b0)

    hs_f, hs_b = pl.pallas_call(
        _scan0_kernel,
        grid=(K,),
        in_specs=[
            pl.BlockSpec((C, B, G4), lambda i: (i, 0, 0)),
            pl.BlockSpec((C, B, G4), lambda i: (K - 1 - i, 0, 1)),
            pl.BlockSpec((HIDDEN, G4), lambda i: (0, 0)),
            pl.BlockSpec((HIDDEN, G4), lambda i: (0, 0)),
        ],
        out_specs=[
            pl.BlockSpec((C, B, HIDDEN), lambda i: (i, 0, 0)),
            pl.BlockSpec((C, B, HIDDEN), lambda i: (K - 1 - i, 0, 0)),
        ],
        out_shape=[
            jax.ShapeDtypeStruct((N, B, HIDDEN), f32),
            jax.ShapeDtypeStruct((N, B, HIDDEN), f32),
        ],
        scratch_shapes=[
            pltpu.VMEM((B, 2 * HIDDEN), f32),
            pltpu.VMEM((B, 2 * HIDDEN), f32),
        ],
    )(g0, g0, whh0f, whh0b)

    g1 = pl.pallas_call(
        _proj1_kernel,
        grid=(K,),
        in_specs=[
            pl.BlockSpec((C, B, HIDDEN), lambda i: (i, 0, 0)),
            pl.BlockSpec((C, B, HIDDEN), lambda i: (i, 0, 0)),
            pl.BlockSpec((HIDDEN, 2 * G4), lambda i: (0, 0)),
            pl.BlockSpec((HIDDEN, 2 * G4), lambda i: (0, 0)),
            pl.BlockSpec((1, 2 * G4), lambda i: (0, 0)),
        ],
        out_specs=pl.BlockSpec((C, B, 2 * G4), lambda i: (i, 0, 0)),
        out_shape=jax.ShapeDtypeStruct((N, B, 2 * G4), f32),
    )(hs_f, hs_b, w1a, w1b, b1)

    out = pl.pallas_call(
        _scan1_kernel,
        grid=(K,),
        in_specs=[
            pl.BlockSpec((C, B, G4), lambda i: (i, 0, 0)),
            pl.BlockSpec((C, B, G4), lambda i: (K - 1 - i, 0, 1)),
            pl.BlockSpec((HIDDEN, G4), lambda i: (0, 0)),
            pl.BlockSpec((HIDDEN, G4), lambda i: (0, 0)),
        ],
        out_specs=pl.BlockSpec((B, 2 * HIDDEN), lambda i: (0, 0)),
        out_shape=jax.ShapeDtypeStruct((B, 2 * HIDDEN), f32),
        scratch_shapes=[
            pltpu.VMEM((B, 2 * HIDDEN), f32),
            pltpu.VMEM((B, 2 * HIDDEN), f32),
        ],
    )(g1, g1, whh1f, whh1b)

    return out


def kernel(x, adj_matrix, params):
    return _run(x, adj_matrix, params)


# speedup vs baseline: 10.1174x; 1.0452x over previous
import functools

import jax
import jax.numpy as jnp
from jax.experimental import pallas as pl
from jax.experimental.pallas import tpu as pltpu

INPUT = 256
HIDDEN = 128
B = 8
N = 1024
BC = 256
C = 128
K = N // C
G4 = 4 * HIDDEN


def _agg_kernel(adj_ref, xf_ref, xr_ref, out_ref):
    m = (adj_ref[0] > 0).astype(jnp.float32)
    nsum = jnp.dot(m, xf_ref[0], preferred_element_type=jnp.float32)
    deg = jnp.sum(m, axis=1, keepdims=True)
    out_ref[0] = (xr_ref[0] + nsum) / (1.0 + deg)


def _proj0_kernel(feat_ref, w_ref, b_ref, out_ref):
    w = w_ref[...]
    b = b_ref[...]
    for bb in range(B):
        out_ref[:, bb, :] = (
            jnp.dot(feat_ref[bb], w, preferred_element_type=jnp.float32) + b
        )


def _proj1_kernel(hf_ref, hb_ref, wa_ref, wb_ref, b_ref, out_ref):
    wa = wa_ref[...]
    wb = wb_ref[...]
    b = b_ref[...]
    for bb in range(B):
        gf = jnp.dot(hf_ref[:, bb, :], wa, preferred_element_type=jnp.float32)
        gb = jnp.dot(hb_ref[:, bb, :], wb, preferred_element_type=jnp.float32)
        out_ref[:, bb, :] = gf + gb + b


def _lstm_cell(gates, c_old):
    i = jax.nn.sigmoid(gates[:, 0 * HIDDEN:1 * HIDDEN])
    f = jax.nn.sigmoid(gates[:, 1 * HIDDEN:2 * HIDDEN])
    g = jnp.tanh(gates[:, 2 * HIDDEN:3 * HIDDEN])
    o = jax.nn.sigmoid(gates[:, 3 * HIDDEN:4 * HIDDEN])
    c_new = f * c_old + i * g
    h_new = o * jnp.tanh(c_new)
    return h_new, c_new


def _scan0_kernel(gf_ref, gb_ref, wf_ref, wb_ref, hsf_ref, hsb_ref,
                  gmf_ref, gmb_ref, c_ref):
    i = pl.program_id(0)

    @pl.when(i == 0)
    def _init():
        gmf_ref[...] = jnp.zeros_like(gmf_ref)
        gmb_ref[...] = jnp.zeros_like(gmb_ref)
        c_ref[...] = jnp.zeros_like(c_ref)

    c0 = c_ref[...]

    def step(j, carry):
        gmf, gmb, cf, cb = carry
        gates_f = gmf + gf_ref[j]
        gates_b = gmb + gb_ref[C - 1 - j]
        hf, cf = _lstm_cell(gates_f, cf)
        hb, cb = _lstm_cell(gates_b, cb)
        hsf_ref[j] = hf
        hsb_ref[C - 1 - j] = hb
        gmf = jnp.dot(hf.astype(jnp.bfloat16), wf_ref[...],
                      preferred_element_type=jnp.float32)
        gmb = jnp.dot(hb.astype(jnp.bfloat16), wb_ref[...],
                      preferred_element_type=jnp.float32)
        return (gmf, gmb, cf, cb)

    gmf, gmb, cf, cb = jax.lax.fori_loop(
        0, C, step,
        (gmf_ref[...], gmb_ref[...], c0[:, :HIDDEN], c0[:, HIDDEN:]))
    gmf_ref[...] = gmf
    gmb_ref[...] = gmb
    c_ref[:, :HIDDEN] = cf
    c_ref[:, HIDDEN:] = cb


def _scan1_kernel(gf_ref, gb_ref, wf_ref, wb_ref, out_ref,
                  gmf_ref, gmb_ref, c_ref):
    i = pl.program_id(0)

    @pl.when(i == 0)
    def _init():
        gmf_ref[...] = jnp.zeros_like(gmf_ref)
        gmb_ref[...] = jnp.zeros_like(gmb_ref)
        c_ref[...] = jnp.zeros_like(c_ref)

    c0 = c_ref[...]

    def step(j, carry):
        gmf, gmb, hf_o, hb_o, cf, cb = carry
        gates_f = gmf + gf_ref[j]
        gates_b = gmb + gb_ref[C - 1 - j]
        hf, cf = _lstm_cell(gates_f, cf)
        hb, cb = _lstm_cell(gates_b, cb)
        gmf = jnp.dot(hf.astype(jnp.bfloat16), wf_ref[...],
                      preferred_element_type=jnp.float32)
        gmb = jnp.dot(hb.astype(jnp.bfloat16), wb_ref[...],
                      preferred_element_type=jnp.float32)
        return (gmf, gmb, hf, hb, cf, cb)

    z = jnp.zeros((B, HIDDEN), jnp.float32)
    gmf, gmb, hf, hb, cf, cb = jax.lax.fori_loop(
        0, C, step,
        (gmf_ref[...], gmb_ref[...], z, z, c0[:, :HIDDEN], c0[:, HIDDEN:]))
    gmf_ref[...] = gmf
    gmb_ref[...] = gmb
    c_ref[:, :HIDDEN] = cf
    c_ref[:, HIDDEN:] = cb
    out_ref[:, :HIDDEN] = hf
    out_ref[:, HIDDEN:] = hb


@jax.jit
def _run(x, adj_matrix, params):
    f32 = jnp.float32
    x = x.astype(f32)

    w0 = jnp.concatenate(
        [params["W_ih_l0_d0"].T, params["W_ih_l0_d1"].T], axis=1)
    b0 = (params["b_ih_l0_d0"] + params["b_hh_l0_d0"],
          params["b_ih_l0_d1"] + params["b_hh_l0_d1"])
    b0 = jnp.concatenate(b0, axis=0).reshape(1, 2 * G4)
    whh0f = params["W_hh_l0_d0"].T.astype(jnp.bfloat16)
    whh0b = params["W_hh_l0_d1"].T.astype(jnp.bfloat16)

    w1 = jnp.concatenate(
        [params["W_ih_l1_d0"].T, params["W_ih_l1_d1"].T], axis=1)
    w1a = w1[:HIDDEN]
    w1b = w1[HIDDEN:]
    b1 = (params["b_ih_l1_d0"] + params["b_hh_l1_d0"],
          params["b_ih_l1_d1"] + params["b_hh_l1_d1"])
    b1 = jnp.concatenate(b1, axis=0).reshape(1, 2 * G4)
    whh1f = params["W_hh_l1_d0"].T.astype(jnp.bfloat16)
    whh1b = params["W_hh_l1_d1"].T.astype(jnp.bfloat16)

    feat = pl.pallas_call(
        _agg_kernel,
        grid=(B, N // BC),
        in_specs=[
            pl.BlockSpec((1, BC, N), lambda b, i: (b, i, 0)),
            pl.BlockSpec((1, N, INPUT), lambda b, i: (b, 0, 0)),
            pl.BlockSpec((1, BC, INPUT), lambda b, i: (b, i, 0)),
        ],
        out_specs=pl.BlockSpec((1, BC, INPUT), lambda b, i: (b, i, 0)),
        out_shape=jax.ShapeDtypeStruct((B, N, INPUT), f32),
    )(adj_matrix, x, x)

    g0 = pl.pallas_call(
        _proj0_kernel,
        grid=(K,),
        in_specs=[
            pl.BlockSpec((B, C, INPUT), lambda i: (0, i, 0)),
            pl.BlockSpec((INPUT, 2 * G4), lambda i: (0, 0)),
            pl.BlockSpec((1, 2 * G4), lambda i: (0, 0)),
        ],
        out_specs=pl.BlockSpec((C, B, 2 * G4), lambda i: (i, 0, 0)),
        out_shape=jax.ShapeDtypeStruct((N, B, 2 * G4), f32),
    )(feat, w0, b0)

    hs_f, hs_b = pl.pallas_call(
        _scan0_kernel,
        grid=(K,),
        in_specs=[
            pl.BlockSpec((C, B, G4), lambda i: (i, 0, 0)),
            pl.BlockSpec((C, B, G4), lambda i: (K - 1 - i, 0, 1)),
            pl.BlockSpec((HIDDEN, G4), lambda i: (0, 0)),
            pl.BlockSpec((HIDDEN, G4), lambda i: (0, 0)),
        ],
        out_specs=[
            pl.BlockSpec((C, B, HIDDEN), lambda i: (i, 0, 0)),
            pl.BlockSpec((C, B, HIDDEN), lambda i: (K - 1 - i, 0, 0)),
        ],
        out_shape=[
            jax.ShapeDtypeStruct((N, B, HIDDEN), f32),
            jax.ShapeDtypeStruct((N, B, HIDDEN), f32),
        ],
        scratch_shapes=[
            pltpu.VMEM((B, G4), f32),
            pltpu.VMEM((B, G4), f32),
            pltpu.VMEM((B, 2 * HIDDEN), f32),
        ],
    )(g0, g0, whh0f, whh0b)

    g1 = pl.pallas_call(
        _proj1_kernel,
        grid=(K,),
        in_specs=[
            pl.BlockSpec((C, B, HIDDEN), lambda i: (i, 0, 0)),
            pl.BlockSpec((C, B, HIDDEN), lambda i: (i, 0, 0)),
            pl.BlockSpec((HIDDEN, 2 * G4), lambda i: (0, 0)),
            pl.BlockSpec((HIDDEN, 2 * G4), lambda i: (0, 0)),
            pl.BlockSpec((1, 2 * G4), lambda i: (0, 0)),
        ],
        out_specs=pl.BlockSpec((C, B, 2 * G4), lambda i: (i, 0, 0)),
        out_shape=jax.ShapeDtypeStruct((N, B, 2 * G4), f32),
    )(hs_f, hs_b, w1a, w1b, b1)

    out = pl.pallas_call(
        _scan1_kernel,
        grid=(K,),
        in_specs=[
            pl.BlockSpec((C, B, G4), lambda i: (i, 0, 0)),
            pl.BlockSpec((C, B, G4), lambda i: (K - 1 - i, 0, 1)),
            pl.BlockSpec((HIDDEN, G4), lambda i: (0, 0)),
            pl.BlockSpec((HIDDEN, G4), lambda i: (0, 0)),
        ],
        out_specs=pl.BlockSpec((B, 2 * HIDDEN), lambda i: (0, 0)),
        out_shape=jax.ShapeDtypeStruct((B, 2 * HIDDEN), f32),
        scratch_shapes=[
            pltpu.VMEM((B, G4), f32),
            pltpu.VMEM((B, G4), f32),
            pltpu.VMEM((B, 2 * HIDDEN), f32),
        ],
    )(g1, g1, whh1f, whh1b)

    return out


def kernel(x, adj_matrix, params):
    return _run(x, adj_matrix, params)


# speedup vs baseline: 11.1142x; 1.0985x over previous
import functools

import jax
import jax.numpy as jnp
from jax.experimental import pallas as pl
from jax.experimental.pallas import tpu as pltpu

INPUT = 256
HIDDEN = 128
B = 8
N = 1024
BC = 256
C = 128
K = N // C
G4 = 4 * HIDDEN


def _agg_kernel(adj_ref, xf_ref, xr_ref, out_ref):
    m = (adj_ref[0] > 0).astype(jnp.float32)
    nsum = jnp.dot(m, xf_ref[0], preferred_element_type=jnp.float32)
    deg = jnp.sum(m, axis=1, keepdims=True)
    out_ref[0] = (xr_ref[0] + nsum) / (1.0 + deg)


def _proj0_kernel(feat_ref, w_ref, b_ref, out_ref):
    w = w_ref[...]
    b = b_ref[...]
    for bb in range(B):
        out_ref[:, bb, :] = (
            jnp.dot(feat_ref[bb], w, preferred_element_type=jnp.float32) + b
        )


def _proj1_kernel(hf_ref, hb_ref, wa_ref, wb_ref, b_ref, out_ref):
    wa = wa_ref[...]
    wb = wb_ref[...]
    b = b_ref[...]
    for bb in range(B):
        gf = jnp.dot(hf_ref[:, bb, :], wa, preferred_element_type=jnp.float32)
        gb = jnp.dot(hb_ref[:, bb, :], wb, preferred_element_type=jnp.float32)
        out_ref[:, bb, :] = gf + gb + b


def _lstm_cell(gates, c_old):
    i = jax.nn.sigmoid(gates[:, 0 * HIDDEN:1 * HIDDEN])
    f = jax.nn.sigmoid(gates[:, 1 * HIDDEN:2 * HIDDEN])
    g = jnp.tanh(gates[:, 2 * HIDDEN:3 * HIDDEN])
    o = jax.nn.sigmoid(gates[:, 3 * HIDDEN:4 * HIDDEN])
    c_new = f * c_old + i * g
    h_new = o * jnp.tanh(c_new)
    return h_new, c_new


def _scan0_kernel(gf_ref, gb_ref, wf_ref, wb_ref, hsf_ref, hsb_ref,
                  gmf_ref, gmb_ref, c_ref):
    i = pl.program_id(0)

    @pl.when(i == 0)
    def _init():
        gmf_ref[...] = jnp.zeros_like(gmf_ref)
        gmb_ref[...] = jnp.zeros_like(gmb_ref)
        c_ref[...] = jnp.zeros_like(c_ref)

    c0 = c_ref[...]

    def step(j, carry):
        gmf, gmb, cf, cb = carry
        gates_f = gmf + gf_ref[j]
        gates_b = gmb + gb_ref[C - 1 - j]
        hf, cf = _lstm_cell(gates_f, cf)
        hb, cb = _lstm_cell(gates_b, cb)
        hsf_ref[j] = hf
        hsb_ref[C - 1 - j] = hb
        gmf = jnp.dot(hf.astype(jnp.bfloat16), wf_ref[...],
                      preferred_element_type=jnp.float32)
        gmb = jnp.dot(hb.astype(jnp.bfloat16), wb_ref[...],
                      preferred_element_type=jnp.float32)
        return (gmf, gmb, cf, cb)

    def step4(j, carry):
        for u in range(4):
            carry = step(4 * j + u, carry)
        return carry

    gmf, gmb, cf, cb = jax.lax.fori_loop(
        0, C // 4, step4,
        (gmf_ref[...], gmb_ref[...], c0[:, :HIDDEN], c0[:, HIDDEN:]))
    gmf_ref[...] = gmf
    gmb_ref[...] = gmb
    c_ref[:, :HIDDEN] = cf
    c_ref[:, HIDDEN:] = cb


def _scan1_kernel(gf_ref, gb_ref, wf_ref, wb_ref, out_ref,
                  gmf_ref, gmb_ref, c_ref):
    i = pl.program_id(0)

    @pl.when(i == 0)
    def _init():
        gmf_ref[...] = jnp.zeros_like(gmf_ref)
        gmb_ref[...] = jnp.zeros_like(gmb_ref)
        c_ref[...] = jnp.zeros_like(c_ref)

    c0 = c_ref[...]

    def step(j, carry):
        gmf, gmb, hf_o, hb_o, cf, cb = carry
        gates_f = gmf + gf_ref[j]
        gates_b = gmb + gb_ref[C - 1 - j]
        hf, cf = _lstm_cell(gates_f, cf)
        hb, cb = _lstm_cell(gates_b, cb)
        gmf = jnp.dot(hf.astype(jnp.bfloat16), wf_ref[...],
                      preferred_element_type=jnp.float32)
        gmb = jnp.dot(hb.astype(jnp.bfloat16), wb_ref[...],
                      preferred_element_type=jnp.float32)
        return (gmf, gmb, hf, hb, cf, cb)

    z = jnp.zeros((B, HIDDEN), jnp.float32)
    def step4(j, carry):
        for u in range(4):
            carry = step(4 * j + u, carry)
        return carry

    gmf, gmb, hf, hb, cf, cb = jax.lax.fori_loop(
        0, C // 4, step4,
        (gmf_ref[...], gmb_ref[...], z, z, c0[:, :HIDDEN], c0[:, HIDDEN:]))
    gmf_ref[...] = gmf
    gmb_ref[...] = gmb
    c_ref[:, :HIDDEN] = cf
    c_ref[:, HIDDEN:] = cb
    out_ref[:, :HIDDEN] = hf
    out_ref[:, HIDDEN:] = hb


@jax.jit
def _run(x, adj_matrix, params):
    f32 = jnp.float32
    x = x.astype(f32)

    w0 = jnp.concatenate(
        [params["W_ih_l0_d0"].T, params["W_ih_l0_d1"].T], axis=1)
    b0 = (params["b_ih_l0_d0"] + params["b_hh_l0_d0"],
          params["b_ih_l0_d1"] + params["b_hh_l0_d1"])
    b0 = jnp.concatenate(b0, axis=0).reshape(1, 2 * G4)
    whh0f = params["W_hh_l0_d0"].T.astype(jnp.bfloat16)
    whh0b = params["W_hh_l0_d1"].T.astype(jnp.bfloat16)

    w1 = jnp.concatenate(
        [params["W_ih_l1_d0"].T, params["W_ih_l1_d1"].T], axis=1)
    w1a = w1[:HIDDEN]
    w1b = w1[HIDDEN:]
    b1 = (params["b_ih_l1_d0"] + params["b_hh_l1_d0"],
          params["b_ih_l1_d1"] + params["b_hh_l1_d1"])
    b1 = jnp.concatenate(b1, axis=0).reshape(1, 2 * G4)
    whh1f = params["W_hh_l1_d0"].T.astype(jnp.bfloat16)
    whh1b = params["W_hh_l1_d1"].T.astype(jnp.bfloat16)

    feat = pl.pallas_call(
        _agg_kernel,
        grid=(B, N // BC),
        in_specs=[
            pl.BlockSpec((1, BC, N), lambda b, i: (b, i, 0)),
            pl.BlockSpec((1, N, INPUT), lambda b, i: (b, 0, 0)),
            pl.BlockSpec((1, BC, INPUT), lambda b, i: (b, i, 0)),
        ],
        out_specs=pl.BlockSpec((1, BC, INPUT), lambda b, i: (b, i, 0)),
        out_shape=jax.ShapeDtypeStruct((B, N, INPUT), f32),
    )(adj_matrix, x, x)

    g0 = pl.pallas_call(
        _proj0_kernel,
        grid=(K,),
        in_specs=[
            pl.BlockSpec((B, C, INPUT), lambda i: (0, i, 0)),
            pl.BlockSpec((INPUT, 2 * G4), lambda i: (0, 0)),
            pl.BlockSpec((1, 2 * G4), lambda i: (0, 0)),
        ],
        out_specs=pl.BlockSpec((C, B, 2 * G4), lambda i: (i, 0, 0)),
        out_shape=jax.ShapeDtypeStruct((N, B, 2 * G4), f32),
    )(feat, w0, b0)

    hs_f, hs_b = pl.pallas_call(
        _scan0_kernel,
        grid=(K,),
        in_specs=[
            pl.BlockSpec((C, B, G4), lambda i: (i, 0, 0)),
            pl.BlockSpec((C, B, G4), lambda i: (K - 1 - i, 0, 1)),
            pl.BlockSpec((HIDDEN, G4), lambda i: (0, 0)),
            pl.BlockSpec((HIDDEN, G4), lambda i: (0, 0)),
        ],
        out_specs=[
            pl.BlockSpec((C, B, HIDDEN), lambda i: (i, 0, 0)),
            pl.BlockSpec((C, B, HIDDEN), lambda i: (K - 1 - i, 0, 0)),
        ],
        out_shape=[
            jax.ShapeDtypeStruct((N, B, HIDDEN), f32),
            jax.ShapeDtypeStruct((N, B, HIDDEN), f32),
        ],
        scratch_shapes=[
            pltpu.VMEM((B, G4), f32),
            pltpu.VMEM((B, G4), f32),
            pltpu.VMEM((B, 2 * HIDDEN), f32),
        ],
    )(g0, g0, whh0f, whh0b)

    g1 = pl.pallas_call(
        _proj1_kernel,
        grid=(K,),
        in_specs=[
            pl.BlockSpec((C, B, HIDDEN), lambda i: (i, 0, 0)),
            pl.BlockSpec((C, B, HIDDEN), lambda i: (i, 0, 0)),
            pl.BlockSpec((HIDDEN, 2 * G4), lambda i: (0, 0)),
            pl.BlockSpec((HIDDEN, 2 * G4), lambda i: (0, 0)),
            pl.BlockSpec((1, 2 * G4), lambda i: (0, 0)),
        ],
        out_specs=pl.BlockSpec((C, B, 2 * G4), lambda i: (i, 0, 0)),
        out_shape=jax.ShapeDtypeStruct((N, B, 2 * G4), f32),
    )(hs_f, hs_b, w1a, w1b, b1)

    out = pl.pallas_call(
        _scan1_kernel,
        grid=(K,),
        in_specs=[
            pl.BlockSpec((C, B, G4), lambda i: (i, 0, 0)),
            pl.BlockSpec((C, B, G4), lambda i: (K - 1 - i, 0, 1)),
            pl.BlockSpec((HIDDEN, G4), lambda i: (0, 0)),
            pl.BlockSpec((HIDDEN, G4), lambda i: (0, 0)),
        ],
        out_specs=pl.BlockSpec((B, 2 * HIDDEN), lambda i: (0, 0)),
        out_shape=jax.ShapeDtypeStruct((B, 2 * HIDDEN), f32),
        scratch_shapes=[
            pltpu.VMEM((B, G4), f32),
            pltpu.VMEM((B, G4), f32),
            pltpu.VMEM((B, 2 * HIDDEN), f32),
        ],
    )(g1, g1, whh1f, whh1b)

    return out


def kernel(x, adj_matrix, params):
    return _run(x, adj_matrix, params)


# speedup vs baseline: 11.2923x; 1.0160x over previous
import functools

import jax
import jax.numpy as jnp
from jax.experimental import pallas as pl
from jax.experimental.pallas import tpu as pltpu

INPUT = 256
HIDDEN = 128
B = 8
N = 1024
BC = 256
C = 128
K = N // C
G4 = 4 * HIDDEN


def _agg_kernel(adj_ref, xf_ref, xr_ref, out_ref):
    m = (adj_ref[0] > 0).astype(jnp.float32)
    nsum = jnp.dot(m, xf_ref[0], preferred_element_type=jnp.float32)
    deg = jnp.sum(m, axis=1, keepdims=True)
    out_ref[0] = (xr_ref[0] + nsum) / (1.0 + deg)


def _proj0_kernel(feat_ref, w_ref, b_ref, out_ref):
    w = w_ref[...]
    b = b_ref[...]
    for bb in range(B):
        out_ref[:, bb, :] = (
            jnp.dot(feat_ref[bb], w, preferred_element_type=jnp.float32) + b
        )


def _proj1_kernel(hf_ref, hb_ref, wa_ref, wb_ref, b_ref, out_ref):
    wa = wa_ref[...]
    wb = wb_ref[...]
    b = b_ref[...]
    for bb in range(B):
        gf = jnp.dot(hf_ref[:, bb, :], wa, preferred_element_type=jnp.float32)
        gb = jnp.dot(hb_ref[:, bb, :], wb, preferred_element_type=jnp.float32)
        out_ref[:, bb, :] = gf + gb + b


def _lstm_cell(gates, c_old):
    i = jax.nn.sigmoid(gates[:, 0 * HIDDEN:1 * HIDDEN])
    f = jax.nn.sigmoid(gates[:, 1 * HIDDEN:2 * HIDDEN])
    g = jnp.tanh(gates[:, 2 * HIDDEN:3 * HIDDEN])
    o = jax.nn.sigmoid(gates[:, 3 * HIDDEN:4 * HIDDEN])
    c_new = f * c_old + i * g
    h_new = o * jnp.tanh(c_new)
    return h_new, c_new


def _scan0_kernel(gf_ref, gb_ref, wf_ref, wb_ref, hsf_ref, hsb_ref,
                  gmf_ref, gmb_ref, c_ref):
    i = pl.program_id(0)

    @pl.when(i == 0)
    def _init():
        gmf_ref[...] = jnp.zeros_like(gmf_ref)
        gmb_ref[...] = jnp.zeros_like(gmb_ref)
        c_ref[...] = jnp.zeros_like(c_ref)

    c0 = c_ref[...]

    def step(j, carry):
        gmf, gmb, cf, cb = carry
        gates_f = gmf + gf_ref[j]
        gates_b = gmb + gb_ref[C - 1 - j]
        hf, cf = _lstm_cell(gates_f, cf)
        hb, cb = _lstm_cell(gates_b, cb)
        hsf_ref[j] = hf
        hsb_ref[C - 1 - j] = hb
        gmf = jnp.dot(hf.astype(jnp.bfloat16), wf_ref[...],
                      preferred_element_type=jnp.float32)
        gmb = jnp.dot(hb.astype(jnp.bfloat16), wb_ref[...],
                      preferred_element_type=jnp.float32)
        return (gmf, gmb, cf, cb)

    def step4(j, carry):
        for u in range(8):
            carry = step(8 * j + u, carry)
        return carry

    gmf, gmb, cf, cb = jax.lax.fori_loop(
        0, C // 8, step4,
        (gmf_ref[...], gmb_ref[...], c0[:, :HIDDEN], c0[:, HIDDEN:]))
    gmf_ref[...] = gmf
    gmb_ref[...] = gmb
    c_ref[:, :HIDDEN] = cf
    c_ref[:, HIDDEN:] = cb


def _scan1_kernel(gf_ref, gb_ref, wf_ref, wb_ref, out_ref,
                  gmf_ref, gmb_ref, c_ref):
    i = pl.program_id(0)

    @pl.when(i == 0)
    def _init():
        gmf_ref[...] = jnp.zeros_like(gmf_ref)
        gmb_ref[...] = jnp.zeros_like(gmb_ref)
        c_ref[...] = jnp.zeros_like(c_ref)

    c0 = c_ref[...]

    def step(j, carry):
        gmf, gmb, hf_o, hb_o, cf, cb = carry
        gates_f = gmf + gf_ref[j]
        gates_b = gmb + gb_ref[C - 1 - j]
        hf, cf = _lstm_cell(gates_f, cf)
        hb, cb = _lstm_cell(gates_b, cb)
        gmf = jnp.dot(hf.astype(jnp.bfloat16), wf_ref[...],
                      preferred_element_type=jnp.float32)
        gmb = jnp.dot(hb.astype(jnp.bfloat16), wb_ref[...],
                      preferred_element_type=jnp.float32)
        return (gmf, gmb, hf, hb, cf, cb)

    z = jnp.zeros((B, HIDDEN), jnp.float32)
    def step4(j, carry):
        for u in range(8):
            carry = step(8 * j + u, carry)
        return carry

    gmf, gmb, hf, hb, cf, cb = jax.lax.fori_loop(
        0, C // 8, step4,
        (gmf_ref[...], gmb_ref[...], z, z, c0[:, :HIDDEN], c0[:, HIDDEN:]))
    gmf_ref[...] = gmf
    gmb_ref[...] = gmb
    c_ref[:, :HIDDEN] = cf
    c_ref[:, HIDDEN:] = cb
    out_ref[:, :HIDDEN] = hf
    out_ref[:, HIDDEN:] = hb


@jax.jit
def _run(x, adj_matrix, params):
    f32 = jnp.float32
    x = x.astype(f32)

    w0 = jnp.concatenate(
        [params["W_ih_l0_d0"].T, params["W_ih_l0_d1"].T], axis=1)
    b0 = (params["b_ih_l0_d0"] + params["b_hh_l0_d0"],
          params["b_ih_l0_d1"] + params["b_hh_l0_d1"])
    b0 = jnp.concatenate(b0, axis=0).reshape(1, 2 * G4)
    whh0f = params["W_hh_l0_d0"].T.astype(jnp.bfloat16)
    whh0b = params["W_hh_l0_d1"].T.astype(jnp.bfloat16)

    w1 = jnp.concatenate(
        [params["W_ih_l1_d0"].T, params["W_ih_l1_d1"].T], axis=1)
    w1a = w1[:HIDDEN]
    w1b = w1[HIDDEN:]
    b1 = (params["b_ih_l1_d0"] + params["b_hh_l1_d0"],
          params["b_ih_l1_d1"] + params["b_hh_l1_d1"])
    b1 = jnp.concatenate(b1, axis=0).reshape(1, 2 * G4)
    whh1f = params["W_hh_l1_d0"].T.astype(jnp.bfloat16)
    whh1b = params["W_hh_l1_d1"].T.astype(jnp.bfloat16)

    feat = pl.pallas_call(
        _agg_kernel,
        grid=(B, N // BC),
        in_specs=[
            pl.BlockSpec((1, BC, N), lambda b, i: (b, i, 0)),
            pl.BlockSpec((1, N, INPUT), lambda b, i: (b, 0, 0)),
            pl.BlockSpec((1, BC, INPUT), lambda b, i: (b, i, 0)),
        ],
        out_specs=pl.BlockSpec((1, BC, INPUT), lambda b, i: (b, i, 0)),
        out_shape=jax.ShapeDtypeStruct((B, N, INPUT), f32),
    )(adj_matrix, x, x)

    g0 = pl.pallas_call(
        _proj0_kernel,
        grid=(K,),
        in_specs=[
            pl.BlockSpec((B, C, INPUT), lambda i: (0, i, 0)),
            pl.BlockSpec((INPUT, 2 * G4), lambda i: (0, 0)),
            pl.BlockSpec((1, 2 * G4), lambda i: (0, 0)),
        ],
        out_specs=pl.BlockSpec((C, B, 2 * G4), lambda i: (i, 0, 0)),
        out_shape=jax.ShapeDtypeStruct((N, B, 2 * G4), f32),
    )(feat, w0, b0)

    hs_f, hs_b = pl.pallas_call(
        _scan0_kernel,
        grid=(K,),
        in_specs=[
            pl.BlockSpec((C, B, G4), lambda i: (i, 0, 0)),
            pl.BlockSpec((C, B, G4), lambda i: (K - 1 - i, 0, 1)),
            pl.BlockSpec((HIDDEN, G4), lambda i: (0, 0)),
            pl.BlockSpec((HIDDEN, G4), lambda i: (0, 0)),
        ],
        out_specs=[
            pl.BlockSpec((C, B, HIDDEN), lambda i: (i, 0, 0)),
            pl.BlockSpec((C, B, HIDDEN), lambda i: (K - 1 - i, 0, 0)),
        ],
        out_shape=[
            jax.ShapeDtypeStruct((N, B, HIDDEN), f32),
            jax.ShapeDtypeStruct((N, B, HIDDEN), f32),
        ],
        scratch_shapes=[
            pltpu.VMEM((B, G4), f32),
            pltpu.VMEM((B, G4), f32),
            pltpu.VMEM((B, 2 * HIDDEN), f32),
        ],
    )(g0, g0, whh0f, whh0b)

    g1 = pl.pallas_call(
        _proj1_kernel,
        grid=(K,),
        in_specs=[
            pl.BlockSpec((C, B, HIDDEN), lambda i: (i, 0, 0)),
            pl.BlockSpec((C, B, HIDDEN), lambda i: (i, 0, 0)),
            pl.BlockSpec((HIDDEN, 2 * G4), lambda i: (0, 0)),
            pl.BlockSpec((HIDDEN, 2 * G4), lambda i: (0, 0)),
            pl.BlockSpec((1, 2 * G4), lambda i: (0, 0)),
        ],
        out_specs=pl.BlockSpec((C, B, 2 * G4), lambda i: (i, 0, 0)),
        out_shape=jax.ShapeDtypeStruct((N, B, 2 * G4), f32),
    )(hs_f, hs_b, w1a, w1b, b1)

    out = pl.pallas_call(
        _scan1_kernel,
        grid=(K,),
        in_specs=[
            pl.BlockSpec((C, B, G4), lambda i: (i, 0, 0)),
            pl.BlockSpec((C, B, G4), lambda i: (K - 1 - i, 0, 1)),
            pl.BlockSpec((HIDDEN, G4), lambda i: (0, 0)),
            pl.BlockSpec((HIDDEN, G4), lambda i: (0, 0)),
        ],
        out_specs=pl.BlockSpec((B, 2 * HIDDEN), lambda i: (0, 0)),
        out_shape=jax.ShapeDtypeStruct((B, 2 * HIDDEN), f32),
        scratch_shapes=[
            pltpu.VMEM((B, G4), f32),
            pltpu.VMEM((B, G4), f32),
            pltpu.VMEM((B, 2 * HIDDEN), f32),
        ],
    )(g1, g1, whh1f, whh1b)

    return out


def kernel(x, adj_matrix, params):
    return _run(x, adj_matrix, params)


# speedup vs baseline: 11.6541x; 1.0320x over previous
import functools

import jax
import jax.numpy as jnp
from jax.experimental import pallas as pl
from jax.experimental.pallas import tpu as pltpu

INPUT = 256
HIDDEN = 128
B = 8
N = 1024
BC = 256
C = 128
K = N // C
G4 = 4 * HIDDEN


def _agg_kernel(adj_ref, xf_ref, xr_ref, out_ref):
    m = (adj_ref[0] > 0).astype(jnp.float32)
    nsum = jnp.dot(m, xf_ref[0], preferred_element_type=jnp.float32)
    deg = jnp.sum(m, axis=1, keepdims=True)
    out_ref[0] = (xr_ref[0] + nsum) / (1.0 + deg)


def _proj0_kernel(feat_ref, w_ref, b_ref, out_ref):
    w = w_ref[...]
    b = b_ref[...]
    for bb in range(B):
        out_ref[:, bb, :] = (
            jnp.dot(feat_ref[bb], w, preferred_element_type=jnp.float32) + b
        )


def _proj1_kernel(hf_ref, hb_ref, wa_ref, wb_ref, b_ref, out_ref):
    wa = wa_ref[...]
    wb = wb_ref[...]
    b = b_ref[...]
    for bb in range(B):
        gf = jnp.dot(hf_ref[:, bb, :], wa, preferred_element_type=jnp.float32)
        gb = jnp.dot(hb_ref[:, bb, :], wb, preferred_element_type=jnp.float32)
        out_ref[:, bb, :] = gf + gb + b


def _lstm_cell(gates, c_old):
    i = jax.nn.sigmoid(gates[:, 0 * HIDDEN:1 * HIDDEN])
    f = jax.nn.sigmoid(gates[:, 1 * HIDDEN:2 * HIDDEN])
    g = jnp.tanh(gates[:, 2 * HIDDEN:3 * HIDDEN])
    o = jax.nn.sigmoid(gates[:, 3 * HIDDEN:4 * HIDDEN])
    c_new = f * c_old + i * g
    h_new = o * jnp.tanh(c_new)
    return h_new, c_new


def _scan0_kernel(ff_ref, fb_ref, wi_f_ref, wi_b_ref, bi_ref,
                  wf_ref, wb_ref, hsf_ref, hsb_ref,
                  gf_ref, gb_ref, gmf_ref, gmb_ref, c_ref):
    bi = bi_ref[...]
    for bb in range(B):
        gf_ref[:, bb, :] = jnp.dot(
            ff_ref[bb], wi_f_ref[...],
            preferred_element_type=jnp.float32) + bi[:, :G4]
        gb_ref[:, bb, :] = jnp.dot(
            fb_ref[bb], wi_b_ref[...],
            preferred_element_type=jnp.float32) + bi[:, G4:]
    i = pl.program_id(0)

    @pl.when(i == 0)
    def _init():
        gmf_ref[...] = jnp.zeros_like(gmf_ref)
        gmb_ref[...] = jnp.zeros_like(gmb_ref)
        c_ref[...] = jnp.zeros_like(c_ref)

    c0 = c_ref[...]

    def step(j, carry):
        gmf, gmb, cf, cb = carry
        gates_f = gmf + gf_ref[j]
        gates_b = gmb + gb_ref[C - 1 - j]
        hf, cf = _lstm_cell(gates_f, cf)
        hb, cb = _lstm_cell(gates_b, cb)
        hsf_ref[j] = hf
        hsb_ref[C - 1 - j] = hb
        gmf = jnp.dot(hf.astype(jnp.bfloat16), wf_ref[...],
                      preferred_element_type=jnp.float32)
        gmb = jnp.dot(hb.astype(jnp.bfloat16), wb_ref[...],
                      preferred_element_type=jnp.float32)
        return (gmf, gmb, cf, cb)

    def step4(j, carry):
        for u in range(8):
            carry = step(8 * j + u, carry)
        return carry

    gmf, gmb, cf, cb = jax.lax.fori_loop(
        0, C // 8, step4,
        (gmf_ref[...], gmb_ref[...], c0[:, :HIDDEN], c0[:, HIDDEN:]))
    gmf_ref[...] = gmf
    gmb_ref[...] = gmb
    c_ref[:, :HIDDEN] = cf
    c_ref[:, HIDDEN:] = cb


def _scan1_kernel(hf_i_ref, hb_i_ref, hf_r_ref, hb_r_ref,
                  waf_ref, wbf_ref, wab_ref, wbb_ref, bi_ref,
                  wf_ref, wb_ref, out_ref,
                  gf_ref, gb_ref, gmf_ref, gmb_ref, c_ref):
    bi = bi_ref[...]
    for bb in range(B):
        gf_ref[:, bb, :] = (
            jnp.dot(hf_i_ref[:, bb, :], waf_ref[...],
                    preferred_element_type=jnp.float32)
            + jnp.dot(hb_i_ref[:, bb, :], wbf_ref[...],
                      preferred_element_type=jnp.float32) + bi[:, :G4])
        gb_ref[:, bb, :] = (
            jnp.dot(hf_r_ref[:, bb, :], wab_ref[...],
                    preferred_element_type=jnp.float32)
            + jnp.dot(hb_r_ref[:, bb, :], wbb_ref[...],
                      preferred_element_type=jnp.float32) + bi[:, G4:])
    i = pl.program_id(0)

    @pl.when(i == 0)
    def _init():
        gmf_ref[...] = jnp.zeros_like(gmf_ref)
        gmb_ref[...] = jnp.zeros_like(gmb_ref)
        c_ref[...] = jnp.zeros_like(c_ref)

    c0 = c_ref[...]

    def step(j, carry):
        gmf, gmb, hf_o, hb_o, cf, cb = carry
        gates_f = gmf + gf_ref[j]
        gates_b = gmb + gb_ref[C - 1 - j]
        hf, cf = _lstm_cell(gates_f, cf)
        hb, cb = _lstm_cell(gates_b, cb)
        gmf = jnp.dot(hf.astype(jnp.bfloat16), wf_ref[...],
                      preferred_element_type=jnp.float32)
        gmb = jnp.dot(hb.astype(jnp.bfloat16), wb_ref[...],
                      preferred_element_type=jnp.float32)
        return (gmf, gmb, hf, hb, cf, cb)

    z = jnp.zeros((B, HIDDEN), jnp.float32)
    def step4(j, carry):
        for u in range(8):
            carry = step(8 * j + u, carry)
        return carry

    gmf, gmb, hf, hb, cf, cb = jax.lax.fori_loop(
        0, C // 8, step4,
        (gmf_ref[...], gmb_ref[...], z, z, c0[:, :HIDDEN], c0[:, HIDDEN:]))
    gmf_ref[...] = gmf
    gmb_ref[...] = gmb
    c_ref[:, :HIDDEN] = cf
    c_ref[:, HIDDEN:] = cb
    out_ref[:, :HIDDEN] = hf
    out_ref[:, HIDDEN:] = hb


@jax.jit
def _run(x, adj_matrix, params):
    f32 = jnp.float32
    x = x.astype(f32)

    w0 = jnp.concatenate(
        [params["W_ih_l0_d0"].T, params["W_ih_l0_d1"].T], axis=1)
    b0 = (params["b_ih_l0_d0"] + params["b_hh_l0_d0"],
          params["b_ih_l0_d1"] + params["b_hh_l0_d1"])
    b0 = jnp.concatenate(b0, axis=0).reshape(1, 2 * G4)
    whh0f = params["W_hh_l0_d0"].T.astype(jnp.bfloat16)
    whh0b = params["W_hh_l0_d1"].T.astype(jnp.bfloat16)

    w1 = jnp.concatenate(
        [params["W_ih_l1_d0"].T, params["W_ih_l1_d1"].T], axis=1)
    w1a = w1[:HIDDEN]
    w1b = w1[HIDDEN:]
    b1 = (params["b_ih_l1_d0"] + params["b_hh_l1_d0"],
          params["b_ih_l1_d1"] + params["b_hh_l1_d1"])
    b1 = jnp.concatenate(b1, axis=0).reshape(1, 2 * G4)
    whh1f = params["W_hh_l1_d0"].T.astype(jnp.bfloat16)
    whh1b = params["W_hh_l1_d1"].T.astype(jnp.bfloat16)

    feat = pl.pallas_call(
        _agg_kernel,
        grid=(B, N // BC),
        in_specs=[
            pl.BlockSpec((1, BC, N), lambda b, i: (b, i, 0)),
            pl.BlockSpec((1, N, INPUT), lambda b, i: (b, 0, 0)),
            pl.BlockSpec((1, BC, INPUT), lambda b, i: (b, i, 0)),
        ],
        out_specs=pl.BlockSpec((1, BC, INPUT), lambda b, i: (b, i, 0)),
        out_shape=jax.ShapeDtypeStruct((B, N, INPUT), f32),
    )(adj_matrix, x, x)

    hs_f, hs_b = pl.pallas_call(
        _scan0_kernel,
        grid=(K,),
        in_specs=[
            pl.BlockSpec((B, C, INPUT), lambda i: (0, i, 0)),
            pl.BlockSpec((B, C, INPUT), lambda i: (0, K - 1 - i, 0)),
            pl.BlockSpec((INPUT, G4), lambda i: (0, 0)),
            pl.BlockSpec((INPUT, G4), lambda i: (0, 1)),
            pl.BlockSpec((1, 2 * G4), lambda i: (0, 0)),
            pl.BlockSpec((HIDDEN, G4), lambda i: (0, 0)),
            pl.BlockSpec((HIDDEN, G4), lambda i: (0, 0)),
        ],
        out_specs=[
            pl.BlockSpec((C, B, HIDDEN), lambda i: (i, 0, 0)),
            pl.BlockSpec((C, B, HIDDEN), lambda i: (K - 1 - i, 0, 0)),
        ],
        out_shape=[
            jax.ShapeDtypeStruct((N, B, HIDDEN), f32),
            jax.ShapeDtypeStruct((N, B, HIDDEN), f32),
        ],
        scratch_shapes=[
            pltpu.VMEM((C, B, G4), f32),
            pltpu.VMEM((C, B, G4), f32),
            pltpu.VMEM((B, G4), f32),
            pltpu.VMEM((B, G4), f32),
            pltpu.VMEM((B, 2 * HIDDEN), f32),
        ],
    )(feat, feat, w0, w0, b0, whh0f, whh0b)

    out = pl.pallas_call(
        _scan1_kernel,
        grid=(K,),
        in_specs=[
            pl.BlockSpec((C, B, HIDDEN), lambda i: (i, 0, 0)),
            pl.BlockSpec((C, B, HIDDEN), lambda i: (i, 0, 0)),
            pl.BlockSpec((C, B, HIDDEN), lambda i: (K - 1 - i, 0, 0)),
            pl.BlockSpec((C, B, HIDDEN), lambda i: (K - 1 - i, 0, 0)),
            pl.BlockSpec((HIDDEN, G4), lambda i: (0, 0)),
            pl.BlockSpec((HIDDEN, G4), lambda i: (0, 0)),
            pl.BlockSpec((HIDDEN, G4), lambda i: (0, 1)),
            pl.BlockSpec((HIDDEN, G4), lambda i: (0, 1)),
            pl.BlockSpec((1, 2 * G4), lambda i: (0, 0)),
            pl.BlockSpec((HIDDEN, G4), lambda i: (0, 0)),
            pl.BlockSpec((HIDDEN, G4), lambda i: (0, 0)),
        ],
        out_specs=pl.BlockSpec((B, 2 * HIDDEN), lambda i: (0, 0)),
        out_shape=jax.ShapeDtypeStruct((B, 2 * HIDDEN), f32),
        scratch_shapes=[
            pltpu.VMEM((C, B, G4), f32),
            pltpu.VMEM((C, B, G4), f32),
            pltpu.VMEM((B, G4), f32),
            pltpu.VMEM((B, G4), f32),
            pltpu.VMEM((B, 2 * HIDDEN), f32),
        ],
    )(hs_f, hs_b, hs_f, hs_b, w1a, w1b, w1a, w1b, b1, whh1f, whh1b)

    return out


def kernel(x, adj_matrix, params):
    return _run(x, adj_matrix, params)


# speedup vs baseline: 11.6637x; 1.0008x over previous
import functools

import jax
import jax.numpy as jnp
from jax.experimental import pallas as pl
from jax.experimental.pallas import tpu as pltpu

INPUT = 256
HIDDEN = 128
B = 8
N = 1024
BC = 256
C = 128
K = N // C
G4 = 4 * HIDDEN


def _agg_kernel(adj_ref, xf_ref, xr_ref, out_ref):
    m = (adj_ref[0] > 0).astype(jnp.float32)
    nsum = jnp.dot(m, xf_ref[0], preferred_element_type=jnp.float32)
    deg = jnp.sum(m, axis=1, keepdims=True)
    out_ref[0] = ((xr_ref[0] + nsum) / (1.0 + deg)).astype(jnp.bfloat16)


def _proj0_kernel(feat_ref, w_ref, b_ref, out_ref):
    w = w_ref[...]
    b = b_ref[...]
    for bb in range(B):
        out_ref[:, bb, :] = (
            jnp.dot(feat_ref[bb], w, preferred_element_type=jnp.float32) + b
        )


def _proj1_kernel(hf_ref, hb_ref, wa_ref, wb_ref, b_ref, out_ref):
    wa = wa_ref[...]
    wb = wb_ref[...]
    b = b_ref[...]
    for bb in range(B):
        gf = jnp.dot(hf_ref[:, bb, :], wa, preferred_element_type=jnp.float32)
        gb = jnp.dot(hb_ref[:, bb, :], wb, preferred_element_type=jnp.float32)
        out_ref[:, bb, :] = gf + gb + b


def _lstm_cell(gates, c_old):
    i = jax.nn.sigmoid(gates[:, 0 * HIDDEN:1 * HIDDEN])
    f = jax.nn.sigmoid(gates[:, 1 * HIDDEN:2 * HIDDEN])
    g = jnp.tanh(gates[:, 2 * HIDDEN:3 * HIDDEN])
    o = jax.nn.sigmoid(gates[:, 3 * HIDDEN:4 * HIDDEN])
    c_new = f * c_old + i * g
    h_new = o * jnp.tanh(c_new)
    return h_new, c_new


def _scan0_kernel(ff_ref, fb_ref, wi_f_ref, wi_b_ref, bi_ref,
                  wf_ref, wb_ref, hsf_ref, hsb_ref,
                  gf_ref, gb_ref, gmf_ref, gmb_ref, c_ref):
    bi = bi_ref[...]
    for bb in range(B):
        gf_ref[:, bb, :] = jnp.dot(
            ff_ref[bb], wi_f_ref[...],
            preferred_element_type=jnp.float32) + bi[:, :G4]
        gb_ref[:, bb, :] = jnp.dot(
            fb_ref[bb], wi_b_ref[...],
            preferred_element_type=jnp.float32) + bi[:, G4:]
    i = pl.program_id(0)

    @pl.when(i == 0)
    def _init():
        gmf_ref[...] = jnp.zeros_like(gmf_ref)
        gmb_ref[...] = jnp.zeros_like(gmb_ref)
        c_ref[...] = jnp.zeros_like(c_ref)

    c0 = c_ref[...]

    def step(j, carry):
        gmf, gmb, cf, cb = carry
        gates_f = gmf + gf_ref[j]
        gates_b = gmb + gb_ref[C - 1 - j]
        hf, cf = _lstm_cell(gates_f, cf)
        hb, cb = _lstm_cell(gates_b, cb)
        hsf_ref[j] = hf.astype(jnp.bfloat16)
        hsb_ref[C - 1 - j] = hb.astype(jnp.bfloat16)
        gmf = jnp.dot(hf.astype(jnp.bfloat16), wf_ref[...],
                      preferred_element_type=jnp.float32)
        gmb = jnp.dot(hb.astype(jnp.bfloat16), wb_ref[...],
                      preferred_element_type=jnp.float32)
        return (gmf, gmb, cf, cb)

    def step4(j, carry):
        for u in range(8):
            carry = step(8 * j + u, carry)
        return carry

    gmf, gmb, cf, cb = jax.lax.fori_loop(
        0, C // 8, step4,
        (gmf_ref[...], gmb_ref[...], c0[:, :HIDDEN], c0[:, HIDDEN:]))
    gmf_ref[...] = gmf
    gmb_ref[...] = gmb
    c_ref[:, :HIDDEN] = cf
    c_ref[:, HIDDEN:] = cb


def _scan1_kernel(hf_i_ref, hb_i_ref, hf_r_ref, hb_r_ref,
                  waf_ref, wbf_ref, wab_ref, wbb_ref, bi_ref,
                  wf_ref, wb_ref, out_ref,
                  gf_ref, gb_ref, gmf_ref, gmb_ref, c_ref):
    bi = bi_ref[...]
    for bb in range(B):
        gf_ref[:, bb, :] = (
            jnp.dot(hf_i_ref[:, bb, :], waf_ref[...],
                    preferred_element_type=jnp.float32)
            + jnp.dot(hb_i_ref[:, bb, :], wbf_ref[...],
                      preferred_element_type=jnp.float32) + bi[:, :G4])
        gb_ref[:, bb, :] = (
            jnp.dot(hf_r_ref[:, bb, :], wab_ref[...],
                    preferred_element_type=jnp.float32)
            + jnp.dot(hb_r_ref[:, bb, :], wbb_ref[...],
                      preferred_element_type=jnp.float32) + bi[:, G4:])
    i = pl.program_id(0)

    @pl.when(i == 0)
    def _init():
        gmf_ref[...] = jnp.zeros_like(gmf_ref)
        gmb_ref[...] = jnp.zeros_like(gmb_ref)
        c_ref[...] = jnp.zeros_like(c_ref)

    c0 = c_ref[...]

    def step(j, carry):
        gmf, gmb, hf_o, hb_o, cf, cb = carry
        gates_f = gmf + gf_ref[j]
        gates_b = gmb + gb_ref[C - 1 - j]
        hf, cf = _lstm_cell(gates_f, cf)
        hb, cb = _lstm_cell(gates_b, cb)
        gmf = jnp.dot(hf.astype(jnp.bfloat16), wf_ref[...],
                      preferred_element_type=jnp.float32)
        gmb = jnp.dot(hb.astype(jnp.bfloat16), wb_ref[...],
                      preferred_element_type=jnp.float32)
        return (gmf, gmb, hf, hb, cf, cb)

    z = jnp.zeros((B, HIDDEN), jnp.float32)
    def step4(j, carry):
        for u in range(8):
            carry = step(8 * j + u, carry)
        return carry

    gmf, gmb, hf, hb, cf, cb = jax.lax.fori_loop(
        0, C // 8, step4,
        (gmf_ref[...], gmb_ref[...], z, z, c0[:, :HIDDEN], c0[:, HIDDEN:]))
    gmf_ref[...] = gmf
    gmb_ref[...] = gmb
    c_ref[:, :HIDDEN] = cf
    c_ref[:, HIDDEN:] = cb
    out_ref[:, :HIDDEN] = hf
    out_ref[:, HIDDEN:] = hb


@jax.jit
def _run(x, adj_matrix, params):
    f32 = jnp.float32
    x = x.astype(f32)

    w0 = jnp.concatenate(
        [params["W_ih_l0_d0"].T, params["W_ih_l0_d1"].T],
        axis=1).astype(jnp.bfloat16)
    b0 = (params["b_ih_l0_d0"] + params["b_hh_l0_d0"],
          params["b_ih_l0_d1"] + params["b_hh_l0_d1"])
    b0 = jnp.concatenate(b0, axis=0).reshape(1, 2 * G4)
    whh0f = params["W_hh_l0_d0"].T.astype(jnp.bfloat16)
    whh0b = params["W_hh_l0_d1"].T.astype(jnp.bfloat16)

    w1 = jnp.concatenate(
        [params["W_ih_l1_d0"].T, params["W_ih_l1_d1"].T], axis=1)
    w1a = w1[:HIDDEN].astype(jnp.bfloat16)
    w1b = w1[HIDDEN:].astype(jnp.bfloat16)
    b1 = (params["b_ih_l1_d0"] + params["b_hh_l1_d0"],
          params["b_ih_l1_d1"] + params["b_hh_l1_d1"])
    b1 = jnp.concatenate(b1, axis=0).reshape(1, 2 * G4)
    whh1f = params["W_hh_l1_d0"].T.astype(jnp.bfloat16)
    whh1b = params["W_hh_l1_d1"].T.astype(jnp.bfloat16)

    feat = pl.pallas_call(
        _agg_kernel,
        grid=(B, N // BC),
        in_specs=[
            pl.BlockSpec((1, BC, N), lambda b, i: (b, i, 0)),
            pl.BlockSpec((1, N, INPUT), lambda b, i: (b, 0, 0)),
            pl.BlockSpec((1, BC, INPUT), lambda b, i: (b, i, 0)),
        ],
        out_specs=pl.BlockSpec((1, BC, INPUT), lambda b, i: (b, i, 0)),
        out_shape=jax.ShapeDtypeStruct((B, N, INPUT), jnp.bfloat16),
    )(adj_matrix, x, x)

    hs_f, hs_b = pl.pallas_call(
        _scan0_kernel,
        grid=(K,),
        in_specs=[
            pl.BlockSpec((B, C, INPUT), lambda i: (0, i, 0)),
            pl.BlockSpec((B, C, INPUT), lambda i: (0, K - 1 - i, 0)),
            pl.BlockSpec((INPUT, G4), lambda i: (0, 0)),
            pl.BlockSpec((INPUT, G4), lambda i: (0, 1)),
            pl.BlockSpec((1, 2 * G4), lambda i: (0, 0)),
            pl.BlockSpec((HIDDEN, G4), lambda i: (0, 0)),
            pl.BlockSpec((HIDDEN, G4), lambda i: (0, 0)),
        ],
        out_specs=[
            pl.BlockSpec((C, B, HIDDEN), lambda i: (i, 0, 0)),
            pl.BlockSpec((C, B, HIDDEN), lambda i: (K - 1 - i, 0, 0)),
        ],
        out_shape=[
            jax.ShapeDtypeStruct((N, B, HIDDEN), jnp.bfloat16),
            jax.ShapeDtypeStruct((N, B, HIDDEN), jnp.bfloat16),
        ],
        scratch_shapes=[
            pltpu.VMEM((C, B, G4), f32),
            pltpu.VMEM((C, B, G4), f32),
            pltpu.VMEM((B, G4), f32),
            pltpu.VMEM((B, G4), f32),
            pltpu.VMEM((B, 2 * HIDDEN), f32),
        ],
    )(feat, feat, w0, w0, b0, whh0f, whh0b)

    out = pl.pallas_call(
        _scan1_kernel,
        grid=(K,),
        in_specs=[
            pl.BlockSpec((C, B, HIDDEN), lambda i: (i, 0, 0)),
            pl.BlockSpec((C, B, HIDDEN), lambda i: (i, 0, 0)),
            pl.BlockSpec((C, B, HIDDEN), lambda i: (K - 1 - i, 0, 0)),
            pl.BlockSpec((C, B, HIDDEN), lambda i: (K - 1 - i, 0, 0)),
            pl.BlockSpec((HIDDEN, G4), lambda i: (0, 0)),
            pl.BlockSpec((HIDDEN, G4), lambda i: (0, 0)),
            pl.BlockSpec((HIDDEN, G4), lambda i: (0, 1)),
            pl.BlockSpec((HIDDEN, G4), lambda i: (0, 1)),
            pl.BlockSpec((1, 2 * G4), lambda i: (0, 0)),
            pl.BlockSpec((HIDDEN, G4), lambda i: (0, 0)),
            pl.BlockSpec((HIDDEN, G4), lambda i: (0, 0)),
        ],
        out_specs=pl.BlockSpec((B, 2 * HIDDEN), lambda i: (0, 0)),
        out_shape=jax.ShapeDtypeStruct((B, 2 * HIDDEN), f32),
        scratch_shapes=[
            pltpu.VMEM((C, B, G4), f32),
            pltpu.VMEM((C, B, G4), f32),
            pltpu.VMEM((B, G4), f32),
            pltpu.VMEM((B, G4), f32),
            pltpu.VMEM((B, 2 * HIDDEN), f32),
        ],
    )(hs_f, hs_b, hs_f, hs_b, w1a, w1b, w1a, w1b, b1, whh1f, whh1b)

    return out


def kernel(x, adj_matrix, params):
    return _run(x, adj_matrix, params)


# speedup vs baseline: 11.7552x; 1.0078x over previous
import functools

import jax
import jax.numpy as jnp
from jax.experimental import pallas as pl
from jax.experimental.pallas import tpu as pltpu

INPUT = 256
HIDDEN = 128
B = 8
N = 1024
BC = 256
C = 128
K = N // C
G4 = 4 * HIDDEN


def _agg_kernel(adj_ref, xf_ref, xr_ref, out_ref):
    m = (adj_ref[0] > 0).astype(jnp.float32)
    nsum = jnp.dot(m, xf_ref[0], preferred_element_type=jnp.float32)
    deg = jnp.sum(m, axis=1, keepdims=True)
    out_ref[0] = ((xr_ref[0] + nsum) / (1.0 + deg)).astype(jnp.bfloat16)


def _proj0_kernel(feat_ref, w_ref, b_ref, out_ref):
    w = w_ref[...]
    b = b_ref[...]
    for bb in range(B):
        out_ref[:, bb, :] = (
            jnp.dot(feat_ref[bb], w, preferred_element_type=jnp.float32) + b
        )


def _proj1_kernel(hf_ref, hb_ref, wa_ref, wb_ref, b_ref, out_ref):
    wa = wa_ref[...]
    wb = wb_ref[...]
    b = b_ref[...]
    for bb in range(B):
        gf = jnp.dot(hf_ref[:, bb, :], wa, preferred_element_type=jnp.float32)
        gb = jnp.dot(hb_ref[:, bb, :], wb, preferred_element_type=jnp.float32)
        out_ref[:, bb, :] = gf + gb + b


def _lstm_cell(gates, c_old):
    i = jax.nn.sigmoid(gates[:, 0 * HIDDEN:1 * HIDDEN])
    f = jax.nn.sigmoid(gates[:, 1 * HIDDEN:2 * HIDDEN])
    g = jnp.tanh(gates[:, 2 * HIDDEN:3 * HIDDEN])
    o = jax.nn.sigmoid(gates[:, 3 * HIDDEN:4 * HIDDEN])
    c_new = f * c_old + i * g
    h_new = o * jnp.tanh(c_new)
    return h_new, c_new


def _scan0_kernel(ff_ref, fb_ref, wi_f_ref, wi_b_ref, bi_ref,
                  wf_ref, wb_ref, hsf_ref, hsb_ref,
                  gf_ref, gb_ref, gmf_ref, gmb_ref, c_ref):
    bi = bi_ref[...]
    for bb in range(B):
        gf_ref[:, bb, :] = jnp.dot(
            ff_ref[bb], wi_f_ref[...],
            preferred_element_type=jnp.float32) + bi[:, :G4]
        gb_ref[:, bb, :] = jnp.dot(
            fb_ref[bb], wi_b_ref[...],
            preferred_element_type=jnp.float32) + bi[:, G4:]
    i = pl.program_id(0)

    @pl.when(i == 0)
    def _init():
        gmf_ref[...] = jnp.zeros_like(gmf_ref)
        gmb_ref[...] = jnp.zeros_like(gmb_ref)
        c_ref[...] = jnp.zeros_like(c_ref)

    c0 = c_ref[...]

    def step(j, carry):
        gmf, gmb, cf, cb = carry
        gates_f = gmf + gf_ref[j]
        gates_b = gmb + gb_ref[C - 1 - j]
        hf, cf = _lstm_cell(gates_f, cf)
        hb, cb = _lstm_cell(gates_b, cb)
        hsf_ref[j] = hf.astype(jnp.bfloat16)
        hsb_ref[C - 1 - j] = hb.astype(jnp.bfloat16)
        gmf = jnp.dot(hf.astype(jnp.bfloat16), wf_ref[...],
                      preferred_element_type=jnp.float32)
        gmb = jnp.dot(hb.astype(jnp.bfloat16), wb_ref[...],
                      preferred_element_type=jnp.float32)
        return (gmf, gmb, cf, cb)

    def step4(j, carry):
        for u in range(16):
            carry = step(16 * j + u, carry)
        return carry

    gmf, gmb, cf, cb = jax.lax.fori_loop(
        0, C // 16, step4,
        (gmf_ref[...], gmb_ref[...], c0[:, :HIDDEN], c0[:, HIDDEN:]))
    gmf_ref[...] = gmf
    gmb_ref[...] = gmb
    c_ref[:, :HIDDEN] = cf
    c_ref[:, HIDDEN:] = cb


def _scan1_kernel(hf_i_ref, hb_i_ref, hf_r_ref, hb_r_ref,
                  waf_ref, wbf_ref, wab_ref, wbb_ref, bi_ref,
                  wf_ref, wb_ref, out_ref,
                  gf_ref, gb_ref, gmf_ref, gmb_ref, c_ref):
    bi = bi_ref[...]
    for bb in range(B):
        gf_ref[:, bb, :] = (
            jnp.dot(hf_i_ref[:, bb, :], waf_ref[...],
                    preferred_element_type=jnp.float32)
            + jnp.dot(hb_i_ref[:, bb, :], wbf_ref[...],
                      preferred_element_type=jnp.float32) + bi[:, :G4])
        gb_ref[:, bb, :] = (
            jnp.dot(hf_r_ref[:, bb, :], wab_ref[...],
                    preferred_element_type=jnp.float32)
            + jnp.dot(hb_r_ref[:, bb, :], wbb_ref[...],
                      preferred_element_type=jnp.float32) + bi[:, G4:])
    i = pl.program_id(0)

    @pl.when(i == 0)
    def _init():
        gmf_ref[...] = jnp.zeros_like(gmf_ref)
        gmb_ref[...] = jnp.zeros_like(gmb_ref)
        c_ref[...] = jnp.zeros_like(c_ref)

    c0 = c_ref[...]

    def step(j, carry):
        gmf, gmb, hf_o, hb_o, cf, cb = carry
        gates_f = gmf + gf_ref[j]
        gates_b = gmb + gb_ref[C - 1 - j]
        hf, cf = _lstm_cell(gates_f, cf)
        hb, cb = _lstm_cell(gates_b, cb)
        gmf = jnp.dot(hf.astype(jnp.bfloat16), wf_ref[...],
                      preferred_element_type=jnp.float32)
        gmb = jnp.dot(hb.astype(jnp.bfloat16), wb_ref[...],
                      preferred_element_type=jnp.float32)
        return (gmf, gmb, hf, hb, cf, cb)

    z = jnp.zeros((B, HIDDEN), jnp.float32)
    def step4(j, carry):
        for u in range(16):
            carry = step(16 * j + u, carry)
        return carry

    gmf, gmb, hf, hb, cf, cb = jax.lax.fori_loop(
        0, C // 16, step4,
        (gmf_ref[...], gmb_ref[...], z, z, c0[:, :HIDDEN], c0[:, HIDDEN:]))
    gmf_ref[...] = gmf
    gmb_ref[...] = gmb
    c_ref[:, :HIDDEN] = cf
    c_ref[:, HIDDEN:] = cb
    out_ref[:, :HIDDEN] = hf
    out_ref[:, HIDDEN:] = hb


@jax.jit
def _run(x, adj_matrix, params):
    f32 = jnp.float32
    x = x.astype(f32)

    w0 = jnp.concatenate(
        [params["W_ih_l0_d0"].T, params["W_ih_l0_d1"].T],
        axis=1).astype(jnp.bfloat16)
    b0 = (params["b_ih_l0_d0"] + params["b_hh_l0_d0"],
          params["b_ih_l0_d1"] + params["b_hh_l0_d1"])
    b0 = jnp.concatenate(b0, axis=0).reshape(1, 2 * G4)
    whh0f = params["W_hh_l0_d0"].T.astype(jnp.bfloat16)
    whh0b = params["W_hh_l0_d1"].T.astype(jnp.bfloat16)

    w1 = jnp.concatenate(
        [params["W_ih_l1_d0"].T, params["W_ih_l1_d1"].T], axis=1)
    w1a = w1[:HIDDEN].astype(jnp.bfloat16)
    w1b = w1[HIDDEN:].astype(jnp.bfloat16)
    b1 = (params["b_ih_l1_d0"] + params["b_hh_l1_d0"],
          params["b_ih_l1_d1"] + params["b_hh_l1_d1"])
    b1 = jnp.concatenate(b1, axis=0).reshape(1, 2 * G4)
    whh1f = params["W_hh_l1_d0"].T.astype(jnp.bfloat16)
    whh1b = params["W_hh_l1_d1"].T.astype(jnp.bfloat16)

    feat = pl.pallas_call(
        _agg_kernel,
        grid=(B, N // BC),
        in_specs=[
            pl.BlockSpec((1, BC, N), lambda b, i: (b, i, 0)),
            pl.BlockSpec((1, N, INPUT), lambda b, i: (b, 0, 0)),
            pl.BlockSpec((1, BC, INPUT), lambda b, i: (b, i, 0)),
        ],
        out_specs=pl.BlockSpec((1, BC, INPUT), lambda b, i: (b, i, 0)),
        out_shape=jax.ShapeDtypeStruct((B, N, INPUT), jnp.bfloat16),
    )(adj_matrix, x, x)

    hs_f, hs_b = pl.pallas_call(
        _scan0_kernel,
        grid=(K,),
        in_specs=[
            pl.BlockSpec((B, C, INPUT), lambda i: (0, i, 0)),
            pl.BlockSpec((B, C, INPUT), lambda i: (0, K - 1 - i, 0)),
            pl.BlockSpec((INPUT, G4), lambda i: (0, 0)),
            pl.BlockSpec((INPUT, G4), lambda i: (0, 1)),
            pl.BlockSpec((1, 2 * G4), lambda i: (0, 0)),
            pl.BlockSpec((HIDDEN, G4), lambda i: (0, 0)),
            pl.BlockSpec((HIDDEN, G4), lambda i: (0, 0)),
        ],
        out_specs=[
            pl.BlockSpec((C, B, HIDDEN), lambda i: (i, 0, 0)),
            pl.BlockSpec((C, B, HIDDEN), lambda i: (K - 1 - i, 0, 0)),
        ],
        out_shape=[
            jax.ShapeDtypeStruct((N, B, HIDDEN), jnp.bfloat16),
            jax.ShapeDtypeStruct((N, B, HIDDEN), jnp.bfloat16),
        ],
        scratch_shapes=[
            pltpu.VMEM((C, B, G4), f32),
            pltpu.VMEM((C, B, G4), f32),
            pltpu.VMEM((B, G4), f32),
            pltpu.VMEM((B, G4), f32),
            pltpu.VMEM((B, 2 * HIDDEN), f32),
        ],
    )(feat, feat, w0, w0, b0, whh0f, whh0b)

    out = pl.pallas_call(
        _scan1_kernel,
        grid=(K,),
        in_specs=[
            pl.BlockSpec((C, B, HIDDEN), lambda i: (i, 0, 0)),
            pl.BlockSpec((C, B, HIDDEN), lambda i: (i, 0, 0)),
            pl.BlockSpec((C, B, HIDDEN), lambda i: (K - 1 - i, 0, 0)),
            pl.BlockSpec((C, B, HIDDEN), lambda i: (K - 1 - i, 0, 0)),
            pl.BlockSpec((HIDDEN, G4), lambda i: (0, 0)),
            pl.BlockSpec((HIDDEN, G4), lambda i: (0, 0)),
            pl.BlockSpec((HIDDEN, G4), lambda i: (0, 1)),
            pl.BlockSpec((HIDDEN, G4), lambda i: (0, 1)),
            pl.BlockSpec((1, 2 * G4), lambda i: (0, 0)),
            pl.BlockSpec((HIDDEN, G4), lambda i: (0, 0)),
            pl.BlockSpec((HIDDEN, G4), lambda i: (0, 0)),
        ],
        out_specs=pl.BlockSpec((B, 2 * HIDDEN), lambda i: (0, 0)),
        out_shape=jax.ShapeDtypeStruct((B, 2 * HIDDEN), f32),
        scratch_shapes=[
            pltpu.VMEM((C, B, G4), f32),
            pltpu.VMEM((C, B, G4), f32),
            pltpu.VMEM((B, G4), f32),
            pltpu.VMEM((B, G4), f32),
            pltpu.VMEM((B, 2 * HIDDEN), f32),
        ],
    )(hs_f, hs_b, hs_f, hs_b, w1a, w1b, w1a, w1b, b1, whh1f, whh1b)

    return out


def kernel(x, adj_matrix, params):
    return _run(x, adj_matrix, params)


# speedup vs baseline: 11.7662x; 1.0009x over previous
import jax
import jax.numpy as jnp
from jax.experimental import pallas as pl
from jax.experimental.pallas import tpu as pltpu

INPUT = 256
HIDDEN = 128
B = 8
N = 1024
BC = 256
C = 128
K = N // C
G4 = 4 * HIDDEN


def _agg_kernel(adj_ref, xf_ref, xr_ref, out_ref):
    m = (adj_ref[0] > 0).astype(jnp.float32)
    nsum = jnp.dot(m, xf_ref[0], preferred_element_type=jnp.float32)
    deg = jnp.sum(m, axis=1, keepdims=True)
    out_ref[0] = ((xr_ref[0] + nsum) / (1.0 + deg)).astype(jnp.bfloat16)


def _lstm_cell(gates, c_old):
    i = jax.nn.sigmoid(gates[:, 0 * HIDDEN:1 * HIDDEN])
    f = jax.nn.sigmoid(gates[:, 1 * HIDDEN:2 * HIDDEN])
    g = jnp.tanh(gates[:, 2 * HIDDEN:3 * HIDDEN])
    o = jax.nn.sigmoid(gates[:, 3 * HIDDEN:4 * HIDDEN])
    c_new = f * c_old + i * g
    h_new = o * jnp.tanh(c_new)
    return h_new, c_new


def _scan0_kernel(ff_ref, fb_ref, wi_f_ref, wi_b_ref, bi_ref,
                  wf_ref, wb_ref, hsf_ref, hsb_ref,
                  gf_ref, gb_ref, gmf_ref, gmb_ref, c_ref):
    bi = bi_ref[...]
    for bb in range(B):
        gf_ref[:, bb, :] = jnp.dot(
            ff_ref[bb], wi_f_ref[...],
            preferred_element_type=jnp.float32) + bi[:, :G4]
        gb_ref[:, bb, :] = jnp.dot(
            fb_ref[bb], wi_b_ref[...],
            preferred_element_type=jnp.float32) + bi[:, G4:]
    i = pl.program_id(0)

    @pl.when(i == 0)
    def _init():
        gmf_ref[...] = jnp.zeros_like(gmf_ref)
        gmb_ref[...] = jnp.zeros_like(gmb_ref)
        c_ref[...] = jnp.zeros_like(c_ref)

    c0 = c_ref[...]

    def step(j, carry):
        gmf, gmb, cf, cb = carry
        gates_f = gmf + gf_ref[j]
        gates_b = gmb + gb_ref[C - 1 - j]
        hf, cf = _lstm_cell(gates_f, cf)
        hb, cb = _lstm_cell(gates_b, cb)
        hsf_ref[j] = hf.astype(jnp.bfloat16)
        hsb_ref[C - 1 - j] = hb.astype(jnp.bfloat16)
        gmf = jnp.dot(hf.astype(jnp.bfloat16), wf_ref[...],
                      preferred_element_type=jnp.float32)
        gmb = jnp.dot(hb.astype(jnp.bfloat16), wb_ref[...],
                      preferred_element_type=jnp.float32)
        return (gmf, gmb, cf, cb)

    def step4(j, carry):
        for u in range(16):
            carry = step(16 * j + u, carry)
        return carry

    gmf, gmb, cf, cb = jax.lax.fori_loop(
        0, C // 16, step4,
        (gmf_ref[...], gmb_ref[...], c0[:, :HIDDEN], c0[:, HIDDEN:]))
    gmf_ref[...] = gmf
    gmb_ref[...] = gmb
    c_ref[:, :HIDDEN] = cf
    c_ref[:, HIDDEN:] = cb


def _scan1_kernel(hf_i_ref, hb_i_ref, hf_r_ref, hb_r_ref,
                  waf_ref, wbf_ref, wab_ref, wbb_ref, bi_ref,
                  wf_ref, wb_ref, out_ref,
                  gf_ref, gb_ref, gmf_ref, gmb_ref, c_ref):
    bi = bi_ref[...]
    for bb in range(B):
        gf_ref[:, bb, :] = (
            jnp.dot(hf_i_ref[:, bb, :], waf_ref[...],
                    preferred_element_type=jnp.float32)
            + jnp.dot(hb_i_ref[:, bb, :], wbf_ref[...],
                      preferred_element_type=jnp.float32) + bi[:, :G4])
        gb_ref[:, bb, :] = (
            jnp.dot(hf_r_ref[:, bb, :], wab_ref[...],
                    preferred_element_type=jnp.float32)
            + jnp.dot(hb_r_ref[:, bb, :], wbb_ref[...],
                      preferred_element_type=jnp.float32) + bi[:, G4:])
    i = pl.program_id(0)

    @pl.when(i == 0)
    def _init():
        gmf_ref[...] = jnp.zeros_like(gmf_ref)
        gmb_ref[...] = jnp.zeros_like(gmb_ref)
        c_ref[...] = jnp.zeros_like(c_ref)

    c0 = c_ref[...]

    def step(j, carry):
        gmf, gmb, hf_o, hb_o, cf, cb = carry
        gates_f = gmf + gf_ref[j]
        gates_b = gmb + gb_ref[C - 1 - j]
        hf, cf = _lstm_cell(gates_f, cf)
        hb, cb = _lstm_cell(gates_b, cb)
        gmf = jnp.dot(hf.astype(jnp.bfloat16), wf_ref[...],
                      preferred_element_type=jnp.float32)
        gmb = jnp.dot(hb.astype(jnp.bfloat16), wb_ref[...],
                      preferred_element_type=jnp.float32)
        return (gmf, gmb, hf, hb, cf, cb)

    z = jnp.zeros((B, HIDDEN), jnp.float32)
    def step4(j, carry):
        for u in range(16):
            carry = step(16 * j + u, carry)
        return carry

    gmf, gmb, hf, hb, cf, cb = jax.lax.fori_loop(
        0, C // 16, step4,
        (gmf_ref[...], gmb_ref[...], z, z, c0[:, :HIDDEN], c0[:, HIDDEN:]))
    gmf_ref[...] = gmf
    gmb_ref[...] = gmb
    c_ref[:, :HIDDEN] = cf
    c_ref[:, HIDDEN:] = cb
    out_ref[:, :HIDDEN] = hf
    out_ref[:, HIDDEN:] = hb


@jax.jit
def _run(x, adj_matrix, params):
    f32 = jnp.float32
    x = x.astype(f32)

    w0 = jnp.concatenate(
        [params["W_ih_l0_d0"].T, params["W_ih_l0_d1"].T],
        axis=1).astype(jnp.bfloat16)
    b0 = (params["b_ih_l0_d0"] + params["b_hh_l0_d0"],
          params["b_ih_l0_d1"] + params["b_hh_l0_d1"])
    b0 = jnp.concatenate(b0, axis=0).reshape(1, 2 * G4)
    whh0f = params["W_hh_l0_d0"].T.astype(jnp.bfloat16)
    whh0b = params["W_hh_l0_d1"].T.astype(jnp.bfloat16)

    w1 = jnp.concatenate(
        [params["W_ih_l1_d0"].T, params["W_ih_l1_d1"].T], axis=1)
    w1a = w1[:HIDDEN].astype(jnp.bfloat16)
    w1b = w1[HIDDEN:].astype(jnp.bfloat16)
    b1 = (params["b_ih_l1_d0"] + params["b_hh_l1_d0"],
          params["b_ih_l1_d1"] + params["b_hh_l1_d1"])
    b1 = jnp.concatenate(b1, axis=0).reshape(1, 2 * G4)
    whh1f = params["W_hh_l1_d0"].T.astype(jnp.bfloat16)
    whh1b = params["W_hh_l1_d1"].T.astype(jnp.bfloat16)

    feat = pl.pallas_call(
        _agg_kernel,
        grid=(B, N // BC),
        in_specs=[
            pl.BlockSpec((1, BC, N), lambda b, i: (b, i, 0)),
            pl.BlockSpec((1, N, INPUT), lambda b, i: (b, 0, 0)),
            pl.BlockSpec((1, BC, INPUT), lambda b, i: (b, i, 0)),
        ],
        out_specs=pl.BlockSpec((1, BC, INPUT), lambda b, i: (b, i, 0)),
        out_shape=jax.ShapeDtypeStruct((B, N, INPUT), jnp.bfloat16),
    )(adj_matrix, x, x)

    hs_f, hs_b = pl.pallas_call(
        _scan0_kernel,
        grid=(K,),
        in_specs=[
            pl.BlockSpec((B, C, INPUT), lambda i: (0, i, 0)),
            pl.BlockSpec((B, C, INPUT), lambda i: (0, K - 1 - i, 0)),
            pl.BlockSpec((INPUT, G4), lambda i: (0, 0)),
            pl.BlockSpec((INPUT, G4), lambda i: (0, 1)),
            pl.BlockSpec((1, 2 * G4), lambda i: (0, 0)),
            pl.BlockSpec((HIDDEN, G4), lambda i: (0, 0)),
            pl.BlockSpec((HIDDEN, G4), lambda i: (0, 0)),
        ],
        out_specs=[
            pl.BlockSpec((C, B, HIDDEN), lambda i: (i, 0, 0)),
            pl.BlockSpec((C, B, HIDDEN), lambda i: (K - 1 - i, 0, 0)),
        ],
        out_shape=[
            jax.ShapeDtypeStruct((N, B, HIDDEN), jnp.bfloat16),
            jax.ShapeDtypeStruct((N, B, HIDDEN), jnp.bfloat16),
        ],
        scratch_shapes=[
            pltpu.VMEM((C, B, G4), f32),
            pltpu.VMEM((C, B, G4), f32),
            pltpu.VMEM((B, G4), f32),
            pltpu.VMEM((B, G4), f32),
            pltpu.VMEM((B, 2 * HIDDEN), f32),
        ],
    )(feat, feat, w0, w0, b0, whh0f, whh0b)

    out = pl.pallas_call(
        _scan1_kernel,
        grid=(K,),
        in_specs=[
            pl.BlockSpec((C, B, HIDDEN), lambda i: (i, 0, 0)),
            pl.BlockSpec((C, B, HIDDEN), lambda i: (i, 0, 0)),
            pl.BlockSpec((C, B, HIDDEN), lambda i: (K - 1 - i, 0, 0)),
            pl.BlockSpec((C, B, HIDDEN), lambda i: (K - 1 - i, 0, 0)),
            pl.BlockSpec((HIDDEN, G4), lambda i: (0, 0)),
            pl.BlockSpec((HIDDEN, G4), lambda i: (0, 0)),
            pl.BlockSpec((HIDDEN, G4), lambda i: (0, 1)),
            pl.BlockSpec((HIDDEN, G4), lambda i: (0, 1)),
            pl.BlockSpec((1, 2 * G4), lambda i: (0, 0)),
            pl.BlockSpec((HIDDEN, G4), lambda i: (0, 0)),
            pl.BlockSpec((HIDDEN, G4), lambda i: (0, 0)),
        ],
        out_specs=pl.BlockSpec((B, 2 * HIDDEN), lambda i: (0, 0)),
        out_shape=jax.ShapeDtypeStruct((B, 2 * HIDDEN), f32),
        scratch_shapes=[
            pltpu.VMEM((C, B, G4), f32),
            pltpu.VMEM((C, B, G4), f32),
            pltpu.VMEM((B, G4), f32),
            pltpu.VMEM((B, G4), f32),
            pltpu.VMEM((B, 2 * HIDDEN), f32),
        ],
    )(hs_f, hs_b, hs_f, hs_b, w1a, w1b, w1a, w1b, b1, whh1f, whh1b)

    return out


def kernel(x, adj_matrix, params):
    return _run(x, adj_matrix, params)
